# Initial kernel scaffold; baseline (speedup 1.0000x reference)
import jax
import jax.numpy as jnp
from jax.experimental import pallas as pl


def kernel(x, edge_index, edge_attr, batch, params):
    raise NotImplementedError("write your pallas kernel here")


# speedup vs baseline: 2.7833x; 2.7833x over previous
import jax
import jax.numpy as jnp
from jax import lax
from jax.experimental import pallas as pl
from jax.experimental.pallas import tpu as pltpu
from jax.experimental.pallas import tpu_sc as plsc

_N = 10000
_E = 320000
_D = 128
_ED = 16
_G = 64

_NC = 2
_NS = 16
_NW = _NC * _NS
_K = 128
_NCHUNKS = _E // _K
_CHUNKS_PER_W = -(-_NCHUNKS // _NW)
_EXPORT_ROWS = 80
_NPIECES = _N // _EXPORT_ROWS
_PIECES_PER_TILE = -(-_NPIECES // _NS)

_BE = 4000
_BN = 2000
_HIGH = lax.Precision.HIGHEST


def _leaky2(h):
    return jnp.where(h > 0, h, 1e-4 * h)


def _edge_body(ea_ref, w_ref, b_ref, o0_ref, o1_ref):
    z = jnp.dot(ea_ref[...], w_ref[...],
                preferred_element_type=jnp.float32, precision=_HIGH)
    z = z + b_ref[...]
    o0_ref[...] = z[:, :_D]
    o1_ref[...] = z[:, _D:]


def _edge_transform(edge_attr, w_cat, b_cat):
    return pl.pallas_call(
        _edge_body,
        grid=(_E // _BE,),
        in_specs=[
            pl.BlockSpec((_BE, _ED), lambda i: (i, 0)),
            pl.BlockSpec((_ED, 2 * _D), lambda i: (0, 0)),
            pl.BlockSpec((1, 2 * _D), lambda i: (0, 0)),
        ],
        out_specs=[
            pl.BlockSpec((_BE, _D), lambda i: (i, 0)),
            pl.BlockSpec((_BE, _D), lambda i: (i, 0)),
        ],
        out_shape=[jax.ShapeDtypeStruct((_E, _D), jnp.float32)] * 2,
    )(edge_attr, w_cat, b_cat)


def _sc_message_body(src_hbm, dst_hbm, e_hbm, x_hbm, out_hbm,
                     idx_v, dst_v, xe_v, e_v, acc_sh, sem):
    c = lax.axis_index("c")
    s = lax.axis_index("s")
    wid = s * _NC + c

    def zrow(r, carry):
        for cc in range(_D // 16):
            xe_v[r, pl.ds(cc * 16, 16)] = jnp.zeros((16,), jnp.float32)
        return carry
    lax.fori_loop(0, _EXPORT_ROWS, zrow, 0)
    for p in range(_PIECES_PER_TILE):
        piece = s + _NS * p

        @pl.when(piece < _NPIECES)
        def _():
            r0 = piece * _EXPORT_ROWS
            pltpu.sync_copy(xe_v.at[pl.ds(0, _EXPORT_ROWS)],
                            acc_sh.at[pl.ds(r0, _EXPORT_ROWS)])
    plsc.subcore_barrier()

    def chunk_body(j, carry):
        chunk = wid + _NW * j

        @pl.when(chunk < _NCHUNKS)
        def _():
            base = chunk * _K
            pltpu.sync_copy(src_hbm.at[pl.ds(base, _K)], idx_v)
            pltpu.sync_copy(dst_hbm.at[pl.ds(base, _K)], dst_v)
            gcp = pltpu.async_copy(x_hbm.at[idx_v], xe_v, sem)
            pltpu.sync_copy(e_hbm.at[pl.ds(base, _K)], e_v)
            gcp.wait()

            def row(r, rcarry):
                for cc in range(_D // 16):
                    sl = pl.ds(cc * 16, 16)
                    xe_v[r, sl] = jnp.maximum(xe_v[r, sl] + e_v[r, sl], 0.0)
                return rcarry
            lax.fori_loop(0, _K, row, 0)

            pltpu.sync_copy(xe_v, acc_sh.at[dst_v], add=True)
        return carry
    lax.fori_loop(0, _CHUNKS_PER_W, chunk_body, 0)
    plsc.subcore_barrier()

    for p in range(_PIECES_PER_TILE):
        piece = s + _NS * p

        @pl.when(piece < _NPIECES)
        def _():
            r0 = piece * _EXPORT_ROWS
            pltpu.sync_copy(acc_sh.at[pl.ds(r0, _EXPORT_ROWS)],
                            xe_v.at[pl.ds(0, _EXPORT_ROWS)])
            pltpu.sync_copy(xe_v.at[pl.ds(0, _EXPORT_ROWS)],
                            out_hbm.at[c, pl.ds(r0, _EXPORT_ROWS)])


def _sc_message(src, dst, e, x):
    mesh = plsc.VectorSubcoreMesh(core_axis_name="c", subcore_axis_name="s")
    f = pl.kernel(
        _sc_message_body,
        out_type=jax.ShapeDtypeStruct((_NC, _N, _D), jnp.float32),
        mesh=mesh,
        scratch_types=[
            pltpu.VMEM((_K,), jnp.int32),
            pltpu.VMEM((_K,), jnp.int32),
            pltpu.VMEM((_K, _D), jnp.float32),
            pltpu.VMEM((_K, _D), jnp.float32),
            pltpu.VMEM_SHARED((_N, _D), jnp.float32),
            pltpu.SemaphoreType.DMA,
        ],
    )
    return f(src, dst, e, x)


def _node_body(inp_ref, agg_ref, w_ref, sc_ref, bi_ref, eps_ref, o_ref):
    z = eps_ref[0, 0] * inp_ref[...] + agg_ref[0] + agg_ref[1]
    h = jnp.dot(z, w_ref[...], preferred_element_type=jnp.float32,
                precision=_HIGH)
    o_ref[...] = _leaky2(h * sc_ref[...] + bi_ref[...])


def _node_update(inp, agg, w, scale, bias, eps1):
    return pl.pallas_call(
        _node_body,
        grid=(_N // _BN,),
        in_specs=[
            pl.BlockSpec((_BN, _D), lambda i: (i, 0)),
            pl.BlockSpec((_NC, _BN, _D), lambda i: (0, i, 0)),
            pl.BlockSpec((_D, _D), lambda i: (0, 0)),
            pl.BlockSpec((1, _D), lambda i: (0, 0)),
            pl.BlockSpec((1, _D), lambda i: (0, 0)),
            pl.BlockSpec(memory_space=pltpu.SMEM),
        ],
        out_specs=pl.BlockSpec((_BN, _D), lambda i: (i, 0)),
        out_shape=jax.ShapeDtypeStruct((_N, _D), jnp.float32),
    )(inp, agg, w, scale, bias, eps1)


def _pool_body(b_ref, h_ref, o_ref):
    i = pl.program_id(0)
    bids = b_ref[0]
    oh = (bids == lax.broadcasted_iota(jnp.int32, (_G, _BN), 0))
    part = jnp.dot(oh.astype(jnp.float32), h_ref[...],
                   preferred_element_type=jnp.float32, precision=_HIGH)

    @pl.when(i == 0)
    def _():
        o_ref[...] = jnp.zeros_like(o_ref)
    o_ref[...] += part


def _pool(batch_row3, h1):
    return pl.pallas_call(
        _pool_body,
        grid=(_N // _BN,),
        in_specs=[
            pl.BlockSpec((1, 1, _BN), lambda i: (i, 0, 0)),
            pl.BlockSpec((_BN, _D), lambda i: (i, 0)),
        ],
        out_specs=pl.BlockSpec((_G, _D), lambda i: (0, 0)),
        out_shape=jax.ShapeDtypeStruct((_G, _D), jnp.float32),
    )(batch_row3, h1)


def _head_body(h0_ref, h1_ref, b_ref, hp_ref, a0_ref, a1_ref, a2_ref,
               cb0_ref, cw1_ref, cb1_ref, fw_ref, fb_ref, o_ref):
    oh = (b_ref[0] == lax.broadcasted_iota(jnp.int32, (_BN, _G), 1))
    hp = jnp.dot(oh.astype(jnp.float32), hp_ref[...],
                 preferred_element_type=jnp.float32, precision=_HIGH)
    z = (jnp.dot(h0_ref[...], a0_ref[...], preferred_element_type=jnp.float32,
                 precision=_HIGH)
         + jnp.dot(h1_ref[...], a1_ref[...], preferred_element_type=jnp.float32,
                   precision=_HIGH)
         + jnp.dot(hp, a2_ref[...], preferred_element_type=jnp.float32,
                   precision=_HIGH)
         + cb0_ref[...])
    z = jnp.where(z > 0, z, 0.01 * z)
    z = jnp.dot(z, cw1_ref[...], preferred_element_type=jnp.float32,
                precision=_HIGH) + cb1_ref[...]
    z = jnp.where(z > 0, z, 0.01 * z)
    y = jnp.dot(z, fw_ref[...], preferred_element_type=jnp.float32,
                precision=_HIGH) + fb_ref[...]
    o_ref[...] = jax.nn.sigmoid(y)


def _head(h0, h1, batch_col3, hp, a0, a1, a2, cb0, cw1, cb1, fw_pad, fb_pad):
    c1 = cw1.shape[0]
    c2 = cw1.shape[1]
    return pl.pallas_call(
        _head_body,
        grid=(_N // _BN,),
        in_specs=[
            pl.BlockSpec((_BN, _D), lambda i: (i, 0)),
            pl.BlockSpec((_BN, _D), lambda i: (i, 0)),
            pl.BlockSpec((1, _BN, 1), lambda i: (i, 0, 0)),
            pl.BlockSpec((_G, _D), lambda i: (0, 0)),
            pl.BlockSpec((_D, c1), lambda i: (0, 0)),
            pl.BlockSpec((_D, c1), lambda i: (0, 0)),
            pl.BlockSpec((_D, c1), lambda i: (0, 0)),
            pl.BlockSpec((1, c1), lambda i: (0, 0)),
            pl.BlockSpec((c1, c2), lambda i: (0, 0)),
            pl.BlockSpec((1, c2), lambda i: (0, 0)),
            pl.BlockSpec((c2, _D), lambda i: (0, 0)),
            pl.BlockSpec((1, _D), lambda i: (0, 0)),
        ],
        out_specs=pl.BlockSpec((_BN, _D), lambda i: (i, 0)),
        out_shape=jax.ShapeDtypeStruct((_N, _D), jnp.float32),
    )(h0, h1, batch_col3, hp, a0, a1, a2, cb0, cw1, cb1, fw_pad, fb_pad)


def kernel(x, edge_index, edge_attr, batch, params):
    src = edge_index[0]
    dst = edge_index[1]

    w_cat = jnp.concatenate([params['eW0'], params['eW1']], axis=1)
    b_cat = jnp.concatenate([params['eb0'], params['eb1']])[None, :]
    e0, e1 = _edge_transform(edge_attr, w_cat, b_cat)

    bn = 1.0 / jnp.sqrt(1.0 + 1e-5)
    s0 = (params['g0'] * bn)[None, :]
    t0 = (params['b0'] * params['g0'] * bn + params['be0'])[None, :]
    s1 = (params['g1'] * bn)[None, :]
    t1 = (params['b1'] * params['g1'] * bn + params['be1'])[None, :]
    eps1_0 = (1.0 + params['eps0']).reshape(1, 1)
    eps1_1 = (1.0 + params['eps1']).reshape(1, 1)

    agg0 = _sc_message(src, dst, e0, x)
    h0 = _node_update(x, agg0, params['W0'], s0, t0, eps1_0)
    agg1 = _sc_message(src, dst, e1, h0)
    h1 = _node_update(h0, agg1, params['W1'], s1, t1, eps1_1)

    batch_row3 = batch.reshape(_N // _BN, 1, _BN)
    batch_col3 = batch.reshape(_N // _BN, _BN, 1)
    hp = _pool(batch_row3, h1)

    a0 = params['cW0'][:_D]
    a1 = params['cW0'][_D:2 * _D]
    a2 = params['cW0'][2 * _D:]
    cb0 = params['cb0'][None, :]
    cb1 = params['cb1'][None, :]
    fw_pad = jnp.concatenate(
        [params['fW'], jnp.zeros((params['fW'].shape[0], _D - 1), jnp.float32)],
        axis=1)
    fb_pad = jnp.concatenate(
        [params['fb'], jnp.zeros((_D - 1,), jnp.float32)])[None, :]

    out_pad = _head(h0, h1, batch_col3, hp, a0, a1, a2, cb0,
                    params['cW1'], cb1, fw_pad, fb_pad)
    return out_pad[:, :1]


# speedup vs baseline: 3.8133x; 1.3701x over previous
import jax
import jax.numpy as jnp
from jax import lax
from jax.experimental import pallas as pl
from jax.experimental.pallas import tpu as pltpu
from jax.experimental.pallas import tpu_sc as plsc

_N = 10000
_E = 320000
_D = 128
_ED = 16
_G = 64

_NC = 2
_NS = 16
_NW = _NC * _NS
_K = 80
_NCHUNKS = _E // _K
_CHUNKS_PER_W = -(-_NCHUNKS // _NW)
_EXPORT_ROWS = 80
_NPIECES = _N // _EXPORT_ROWS
_PIECES_PER_TILE = -(-_NPIECES // _NS)

_BE = 4000
_BN = 2000
_HIGH = lax.Precision.HIGHEST


def _leaky2(h):
    return jnp.where(h > 0, h, 1e-4 * h)


def _edge_body(ea_ref, w_ref, b_ref, o0_ref, o1_ref):
    z = jnp.dot(ea_ref[...], w_ref[...],
                preferred_element_type=jnp.float32)
    z = z + b_ref[...]
    o0_ref[...] = z[:, :_D]
    o1_ref[...] = z[:, _D:]


def _edge_transform(edge_attr, w_cat, b_cat):
    return pl.pallas_call(
        _edge_body,
        grid=(_E // _BE,),
        in_specs=[
            pl.BlockSpec((_BE, _ED), lambda i: (i, 0)),
            pl.BlockSpec((_ED, 2 * _D), lambda i: (0, 0)),
            pl.BlockSpec((1, 2 * _D), lambda i: (0, 0)),
        ],
        out_specs=[
            pl.BlockSpec((_BE, _D), lambda i: (i, 0)),
            pl.BlockSpec((_BE, _D), lambda i: (i, 0)),
        ],
        out_shape=[jax.ShapeDtypeStruct((_E, _D), jnp.float32)] * 2,
    )(edge_attr, w_cat, b_cat)


def _sc_message_body(src_hbm, dst_hbm, e_hbm, x_hbm, out_hbm,
                     idx0, idx1, dstv0, dstv1, xe0, xe1, ev0, ev1,
                     acc_sh, sem_i0, sem_i1, sem_g0, sem_g1):
    c = lax.axis_index("c")
    s = lax.axis_index("s")
    wid = s * _NC + c
    nj = (_NCHUNKS - wid + _NW - 1) // _NW
    slots = ((idx0, dstv0, ev0, xe0, sem_i0, sem_g0),
             (idx1, dstv1, ev1, xe1, sem_i1, sem_g1))
    xe_v = xe0

    def zrow(r, carry):
        for cc in range(_D // 16):
            xe_v[r, pl.ds(cc * 16, 16)] = jnp.zeros((16,), jnp.float32)
        return carry
    lax.fori_loop(0, _EXPORT_ROWS, zrow, 0)
    for p in range(_PIECES_PER_TILE):
        piece = s + _NS * p

        @pl.when(piece < _NPIECES)
        def _():
            r0 = piece * _EXPORT_ROWS
            pltpu.sync_copy(xe_v.at[pl.ds(0, _EXPORT_ROWS)],
                            acc_sh.at[pl.ds(r0, _EXPORT_ROWS)])
    plsc.subcore_barrier()

    def _prefetch(j, slot, guard):
        idx_b, dst_b, e_b, _, sem_i, _ = slot

        @pl.when(guard)
        def _():
            base = (wid + _NW * j) * _K
            pltpu.async_copy(src_hbm.at[pl.ds(base, _K)], idx_b, sem_i)
            pltpu.async_copy(dst_hbm.at[pl.ds(base, _K)], dst_b, sem_i)
            pltpu.async_copy(e_hbm.at[pl.ds(base, _K)], e_b, sem_i)

    def _wait_prefetch(slot, guard):
        idx_b, dst_b, e_b, _, sem_i, _ = slot

        @pl.when(guard)
        def _():
            pltpu.make_async_copy(src_hbm.at[pl.ds(0, _K)], idx_b, sem_i).wait()
            pltpu.make_async_copy(dst_hbm.at[pl.ds(0, _K)], dst_b, sem_i).wait()
            pltpu.make_async_copy(e_hbm.at[pl.ds(0, _K)], e_b, sem_i).wait()

    def _start_gather(slot, guard):
        idx_b, _, _, xe_b, _, sem_g = slot

        @pl.when(guard)
        def _():
            pltpu.async_copy(x_hbm.at[idx_b], xe_b, sem_g)

    def _wait_gather(slot, guard):
        idx_b, _, _, xe_b, _, sem_g = slot

        @pl.when(guard)
        def _():
            pltpu.make_async_copy(x_hbm.at[idx_b], xe_b, sem_g).wait()

    def _compute_scatter(slot, guard):
        _, dst_b, e_b, xe_b, _, _ = slot

        @pl.when(guard)
        def _():
            def row(r, rcarry):
                for cc in range(_D // 16):
                    sl = pl.ds(cc * 16, 16)
                    xe_b[r, sl] = jnp.maximum(xe_b[r, sl] + e_b[r, sl], 0.0)
                return rcarry
            lax.fori_loop(0, _K, row, 0)
            pltpu.sync_copy(xe_b, acc_sh.at[dst_b], add=True)

    _prefetch(0, slots[0], 0 < nj)
    _prefetch(1, slots[1], 1 < nj)
    _wait_prefetch(slots[0], 0 < nj)
    _start_gather(slots[0], 0 < nj)

    def pair_body(p, carry):
        for b in range(2):
            j = 2 * p + b
            o = 1 - b
            _wait_prefetch(slots[o], j + 1 < nj)
            _start_gather(slots[o], j + 1 < nj)
            _wait_gather(slots[b], j < nj)
            _compute_scatter(slots[b], j < nj)
            _prefetch(j + 2, slots[b], j + 2 < nj)
        return carry
    lax.fori_loop(0, (_CHUNKS_PER_W + 1) // 2, pair_body, 0)
    plsc.subcore_barrier()

    for p in range(_PIECES_PER_TILE):
        piece = s + _NS * p

        @pl.when(piece < _NPIECES)
        def _():
            r0 = piece * _EXPORT_ROWS
            pltpu.sync_copy(acc_sh.at[pl.ds(r0, _EXPORT_ROWS)],
                            xe_v.at[pl.ds(0, _EXPORT_ROWS)])
            pltpu.sync_copy(xe_v.at[pl.ds(0, _EXPORT_ROWS)],
                            out_hbm.at[c, pl.ds(r0, _EXPORT_ROWS)])


def _sc_message(src, dst, e, x):
    mesh = plsc.VectorSubcoreMesh(core_axis_name="c", subcore_axis_name="s")
    f = pl.kernel(
        _sc_message_body,
        out_type=jax.ShapeDtypeStruct((_NC, _N, _D), jnp.float32),
        mesh=mesh,
        scratch_types=[
            pltpu.VMEM((_K,), jnp.int32),
            pltpu.VMEM((_K,), jnp.int32),
            pltpu.VMEM((_K,), jnp.int32),
            pltpu.VMEM((_K,), jnp.int32),
            pltpu.VMEM((_K, _D), jnp.float32),
            pltpu.VMEM((_K, _D), jnp.float32),
            pltpu.VMEM((_K, _D), jnp.float32),
            pltpu.VMEM((_K, _D), jnp.float32),
            pltpu.VMEM_SHARED((_N, _D), jnp.float32),
            pltpu.SemaphoreType.DMA,
            pltpu.SemaphoreType.DMA,
            pltpu.SemaphoreType.DMA,
            pltpu.SemaphoreType.DMA,
        ],
    )
    return f(src, dst, e, x)


def _node_body(inp_ref, agg_ref, w_ref, sc_ref, bi_ref, eps_ref, o_ref):
    z = eps_ref[0, 0] * inp_ref[...] + agg_ref[0] + agg_ref[1]
    h = jnp.dot(z, w_ref[...], preferred_element_type=jnp.float32,
                precision=_HIGH)
    o_ref[...] = _leaky2(h * sc_ref[...] + bi_ref[...])


def _node_update(inp, agg, w, scale, bias, eps1):
    return pl.pallas_call(
        _node_body,
        grid=(_N // _BN,),
        in_specs=[
            pl.BlockSpec((_BN, _D), lambda i: (i, 0)),
            pl.BlockSpec((_NC, _BN, _D), lambda i: (0, i, 0)),
            pl.BlockSpec((_D, _D), lambda i: (0, 0)),
            pl.BlockSpec((1, _D), lambda i: (0, 0)),
            pl.BlockSpec((1, _D), lambda i: (0, 0)),
            pl.BlockSpec(memory_space=pltpu.SMEM),
        ],
        out_specs=pl.BlockSpec((_BN, _D), lambda i: (i, 0)),
        out_shape=jax.ShapeDtypeStruct((_N, _D), jnp.float32),
    )(inp, agg, w, scale, bias, eps1)


def _pool_body(b_ref, h_ref, o_ref):
    i = pl.program_id(0)
    bids = b_ref[0]
    oh = (bids == lax.broadcasted_iota(jnp.int32, (_G, _BN), 0))
    part = jnp.dot(oh.astype(jnp.float32), h_ref[...],
                   preferred_element_type=jnp.float32, precision=_HIGH)

    @pl.when(i == 0)
    def _():
        o_ref[...] = jnp.zeros_like(o_ref)
    o_ref[...] += part


def _pool(batch_row3, h1):
    return pl.pallas_call(
        _pool_body,
        grid=(_N // _BN,),
        in_specs=[
            pl.BlockSpec((1, 1, _BN), lambda i: (i, 0, 0)),
            pl.BlockSpec((_BN, _D), lambda i: (i, 0)),
        ],
        out_specs=pl.BlockSpec((_G, _D), lambda i: (0, 0)),
        out_shape=jax.ShapeDtypeStruct((_G, _D), jnp.float32),
    )(batch_row3, h1)


def _head_body(h0_ref, h1_ref, b_ref, hp_ref, a0_ref, a1_ref, a2_ref,
               cb0_ref, cw1_ref, cb1_ref, fw_ref, fb_ref, o_ref):
    oh = (b_ref[0] == lax.broadcasted_iota(jnp.int32, (_BN, _G), 1))
    hp = jnp.dot(oh.astype(jnp.float32), hp_ref[...],
                 preferred_element_type=jnp.float32)
    z = (jnp.dot(h0_ref[...], a0_ref[...], preferred_element_type=jnp.float32)
         + jnp.dot(h1_ref[...], a1_ref[...], preferred_element_type=jnp.float32)
         + jnp.dot(hp, a2_ref[...], preferred_element_type=jnp.float32)
         + cb0_ref[...])
    z = jnp.where(z > 0, z, 0.01 * z)
    z = jnp.dot(z, cw1_ref[...], preferred_element_type=jnp.float32) + cb1_ref[...]
    z = jnp.where(z > 0, z, 0.01 * z)
    y = jnp.dot(z, fw_ref[...], preferred_element_type=jnp.float32) + fb_ref[...]
    o_ref[...] = jax.nn.sigmoid(y)


def _head(h0, h1, batch_col3, hp, a0, a1, a2, cb0, cw1, cb1, fw_pad, fb_pad):
    c1 = cw1.shape[0]
    c2 = cw1.shape[1]
    return pl.pallas_call(
        _head_body,
        grid=(_N // _BN,),
        in_specs=[
            pl.BlockSpec((_BN, _D), lambda i: (i, 0)),
            pl.BlockSpec((_BN, _D), lambda i: (i, 0)),
            pl.BlockSpec((1, _BN, 1), lambda i: (i, 0, 0)),
            pl.BlockSpec((_G, _D), lambda i: (0, 0)),
            pl.BlockSpec((_D, c1), lambda i: (0, 0)),
            pl.BlockSpec((_D, c1), lambda i: (0, 0)),
            pl.BlockSpec((_D, c1), lambda i: (0, 0)),
            pl.BlockSpec((1, c1), lambda i: (0, 0)),
            pl.BlockSpec((c1, c2), lambda i: (0, 0)),
            pl.BlockSpec((1, c2), lambda i: (0, 0)),
            pl.BlockSpec((c2, _D), lambda i: (0, 0)),
            pl.BlockSpec((1, _D), lambda i: (0, 0)),
        ],
        out_specs=pl.BlockSpec((_BN, _D), lambda i: (i, 0)),
        out_shape=jax.ShapeDtypeStruct((_N, _D), jnp.float32),
    )(h0, h1, batch_col3, hp, a0, a1, a2, cb0, cw1, cb1, fw_pad, fb_pad)


def kernel(x, edge_index, edge_attr, batch, params):
    src = edge_index[0]
    dst = edge_index[1]

    w_cat = jnp.concatenate([params['eW0'], params['eW1']], axis=1)
    b_cat = jnp.concatenate([params['eb0'], params['eb1']])[None, :]
    e0, e1 = _edge_transform(edge_attr, w_cat, b_cat)

    bn = 1.0 / jnp.sqrt(1.0 + 1e-5)
    s0 = (params['g0'] * bn)[None, :]
    t0 = (params['b0'] * params['g0'] * bn + params['be0'])[None, :]
    s1 = (params['g1'] * bn)[None, :]
    t1 = (params['b1'] * params['g1'] * bn + params['be1'])[None, :]
    eps1_0 = (1.0 + params['eps0']).reshape(1, 1)
    eps1_1 = (1.0 + params['eps1']).reshape(1, 1)

    agg0 = _sc_message(src, dst, e0, x)
    h0 = _node_update(x, agg0, params['W0'], s0, t0, eps1_0)
    agg1 = _sc_message(src, dst, e1, h0)
    h1 = _node_update(h0, agg1, params['W1'], s1, t1, eps1_1)

    batch_row3 = batch.reshape(_N // _BN, 1, _BN)
    batch_col3 = batch.reshape(_N // _BN, _BN, 1)
    hp = _pool(batch_row3, h1)

    a0 = params['cW0'][:_D]
    a1 = params['cW0'][_D:2 * _D]
    a2 = params['cW0'][2 * _D:]
    cb0 = params['cb0'][None, :]
    cb1 = params['cb1'][None, :]
    fw_pad = jnp.concatenate(
        [params['fW'], jnp.zeros((params['fW'].shape[0], _D - 1), jnp.float32)],
        axis=1)
    fb_pad = jnp.concatenate(
        [params['fb'], jnp.zeros((_D - 1,), jnp.float32)])[None, :]

    out_pad = _head(h0, h1, batch_col3, hp, a0, a1, a2, cb0,
                    params['cW1'], cb1, fw_pad, fb_pad)
    return out_pad[:, :1]


# speedup vs baseline: 4.8820x; 1.2802x over previous
import jax
import jax.numpy as jnp
from jax import lax
from jax.experimental import pallas as pl
from jax.experimental.pallas import tpu as pltpu
from jax.experimental.pallas import tpu_sc as plsc

_N = 10000
_E = 320000
_D = 128
_ED = 16
_G = 64

_NC = 2
_NS = 16
_NW = _NC * _NS
_K = 80
_NCHUNKS = _E // _K
_CHUNKS_PER_W = -(-_NCHUNKS // _NW)
_EXPORT_ROWS = 80
_NPIECES = _N // _EXPORT_ROWS
_PIECES_PER_TILE = -(-_NPIECES // _NS)

_BE = 2560
_BN = 2000
_HIGH = lax.Precision.HIGHEST


def _leaky2(h):
    return jnp.where(h > 0, h, 1e-4 * h)


def _edge_body(ea_ref, w_ref, b_ref, o0_ref, o1_ref):
    z = lax.dot_general(ea_ref[...].astype(jnp.bfloat16),
                        w_ref[...].astype(jnp.bfloat16),
                        (((0,), (0,)), ((), ())),
                        preferred_element_type=jnp.float32)
    z = z + b_ref[...]
    o0_ref[...] = z[:, :_D]
    o1_ref[...] = z[:, _D:]


def _edge_transform(edge_attr_t, w_cat, b_cat):
    return pl.pallas_call(
        _edge_body,
        grid=(_E // _BE,),
        in_specs=[
            pl.BlockSpec((_ED, _BE), lambda i: (0, i)),
            pl.BlockSpec((_ED, 2 * _D), lambda i: (0, 0)),
            pl.BlockSpec((1, 2 * _D), lambda i: (0, 0)),
        ],
        out_specs=[
            pl.BlockSpec((_BE, _D), lambda i: (i, 0)),
            pl.BlockSpec((_BE, _D), lambda i: (i, 0)),
        ],
        out_shape=[jax.ShapeDtypeStruct((_E, _D), jnp.float32)] * 2,
    )(edge_attr_t, w_cat, b_cat)


def _sc_message_body(src_hbm, dst_hbm, e_hbm, x_hbm, out_hbm,
                     idx0, idx1, dstv0, dstv1, dsts0, dsts1, xe0, xe1,
                     ev0, ev1, acc_sh,
                     sem_i0, sem_i1, sem_g0, sem_g1, sem_s0, sem_s1):
    c = lax.axis_index("c")
    s = lax.axis_index("s")
    wid = s * _NC + c
    nj = (_NCHUNKS - wid + _NW - 1) // _NW
    slots = ((idx0, dstv0, dsts0, ev0, xe0, sem_i0, sem_g0, sem_s0),
             (idx1, dstv1, dsts1, ev1, xe1, sem_i1, sem_g1, sem_s1))
    xe_v = xe0

    def zrow(r, carry):
        for cc in range(_D // 16):
            xe_v[r, pl.ds(cc * 16, 16)] = jnp.zeros((16,), jnp.float32)
        return carry
    lax.fori_loop(0, _EXPORT_ROWS, zrow, 0)
    for p in range(_PIECES_PER_TILE):
        piece = s + _NS * p

        @pl.when(piece < _NPIECES)
        def _():
            r0 = piece * _EXPORT_ROWS
            pltpu.sync_copy(xe_v.at[pl.ds(0, _EXPORT_ROWS)],
                            acc_sh.at[pl.ds(r0, _EXPORT_ROWS)])
    plsc.subcore_barrier()

    def _prefetch(j, slot, guard):
        idx_b, dst_b, _, e_b, _, sem_i, _, _ = slot

        @pl.when(guard)
        def _():
            base = (wid + _NW * j) * _K
            pltpu.async_copy(src_hbm.at[pl.ds(base, _K)], idx_b, sem_i)
            pltpu.async_copy(dst_hbm.at[pl.ds(base, _K)], dst_b, sem_i)
            pltpu.async_copy(e_hbm.at[pl.ds(base, _K)], e_b, sem_i)

    def _wait_prefetch(slot, guard):
        idx_b, dst_b, _, e_b, _, sem_i, _, _ = slot

        @pl.when(guard)
        def _():
            pltpu.make_async_copy(src_hbm.at[pl.ds(0, _K)], idx_b, sem_i).wait()
            pltpu.make_async_copy(dst_hbm.at[pl.ds(0, _K)], dst_b, sem_i).wait()
            pltpu.make_async_copy(e_hbm.at[pl.ds(0, _K)], e_b, sem_i).wait()

    def _wait_scatter(slot, guard):
        _, _, dst2_b, _, xe_b, _, _, sem_s = slot

        @pl.when(guard)
        def _():
            pltpu.make_async_copy(xe_b, acc_sh.at[dst2_b], sem_s).wait()

    def _start_gather(slot, guard):
        idx_b, _, _, _, xe_b, _, sem_g, _ = slot

        @pl.when(guard)
        def _():
            pltpu.async_copy(x_hbm.at[idx_b], xe_b, sem_g)

    def _wait_gather(slot, guard):
        idx_b, _, _, _, xe_b, _, sem_g, _ = slot

        @pl.when(guard)
        def _():
            pltpu.make_async_copy(x_hbm.at[idx_b], xe_b, sem_g).wait()

    def _compute_scatter(slot, guard):
        _, dst_b, dst2_b, e_b, xe_b, _, _, sem_s = slot

        @pl.when(guard)
        def _():
            def row(r, rcarry):
                for cc in range(_D // 16):
                    sl = pl.ds(cc * 16, 16)
                    xe_b[r, sl] = jnp.maximum(xe_b[r, sl] + e_b[r, sl], 0.0)
                return rcarry
            lax.fori_loop(0, _K, row, 0)
            for k in range(_K // 16):
                sl = pl.ds(k * 16, 16)
                dst2_b[sl] = dst_b[sl]
            pltpu.async_copy(xe_b, acc_sh.at[dst2_b], sem_s, add=True)

    _prefetch(0, slots[0], 0 < nj)
    _prefetch(1, slots[1], 1 < nj)
    _wait_prefetch(slots[0], 0 < nj)
    _start_gather(slots[0], 0 < nj)

    def pair_body(p, carry):
        for b in range(2):
            j = 2 * p + b
            o = 1 - b
            _wait_prefetch(slots[o], j + 1 < nj)
            _wait_scatter(slots[o], (j >= 1) & (j + 1 < nj))
            _start_gather(slots[o], j + 1 < nj)
            _wait_gather(slots[b], j < nj)
            _compute_scatter(slots[b], j < nj)
            _prefetch(j + 2, slots[b], j + 2 < nj)
        return carry
    lax.fori_loop(0, (_CHUNKS_PER_W + 1) // 2, pair_body, 0)
    _wait_scatter(slots[0], nj >= 1)
    _wait_scatter(slots[1], nj >= 2)
    plsc.subcore_barrier()

    for p in range(_PIECES_PER_TILE):
        piece = s + _NS * p

        @pl.when(piece < _NPIECES)
        def _():
            r0 = piece * _EXPORT_ROWS
            pltpu.sync_copy(acc_sh.at[pl.ds(r0, _EXPORT_ROWS)],
                            xe_v.at[pl.ds(0, _EXPORT_ROWS)])
            pltpu.sync_copy(xe_v.at[pl.ds(0, _EXPORT_ROWS)],
                            out_hbm.at[c, pl.ds(r0, _EXPORT_ROWS)])


def _sc_message(src, dst, e, x):
    mesh = plsc.VectorSubcoreMesh(core_axis_name="c", subcore_axis_name="s")
    f = pl.kernel(
        _sc_message_body,
        out_type=jax.ShapeDtypeStruct((_NC, _N, _D), jnp.float32),
        mesh=mesh,
        scratch_types=(
            [pltpu.VMEM((_K,), jnp.int32)] * 6
            + [pltpu.VMEM((_K, _D), jnp.float32)] * 4
            + [pltpu.VMEM_SHARED((_N, _D), jnp.float32)]
            + [pltpu.SemaphoreType.DMA] * 6
        ),
    )
    return f(src, dst, e, x)


def _node_body(inp_ref, agg_ref, w_ref, sc_ref, bi_ref, eps_ref, o_ref):
    z = eps_ref[0, 0] * inp_ref[...] + agg_ref[0] + agg_ref[1]
    h = jnp.dot(z, w_ref[...], preferred_element_type=jnp.float32,
                precision=_HIGH)
    o_ref[...] = _leaky2(h * sc_ref[...] + bi_ref[...])


def _node_update(inp, agg, w, scale, bias, eps1):
    return pl.pallas_call(
        _node_body,
        grid=(_N // _BN,),
        in_specs=[
            pl.BlockSpec((_BN, _D), lambda i: (i, 0)),
            pl.BlockSpec((_NC, _BN, _D), lambda i: (0, i, 0)),
            pl.BlockSpec((_D, _D), lambda i: (0, 0)),
            pl.BlockSpec((1, _D), lambda i: (0, 0)),
            pl.BlockSpec((1, _D), lambda i: (0, 0)),
            pl.BlockSpec(memory_space=pltpu.SMEM),
        ],
        out_specs=pl.BlockSpec((_BN, _D), lambda i: (i, 0)),
        out_shape=jax.ShapeDtypeStruct((_N, _D), jnp.float32),
    )(inp, agg, w, scale, bias, eps1)


def _pool_body(b_ref, h_ref, o_ref):
    i = pl.program_id(0)
    bids = b_ref[0]
    oh = (bids == lax.broadcasted_iota(jnp.int32, (_G, _BN), 0))
    part = jnp.dot(oh.astype(jnp.float32), h_ref[...],
                   preferred_element_type=jnp.float32, precision=_HIGH)

    @pl.when(i == 0)
    def _():
        o_ref[...] = jnp.zeros_like(o_ref)
    o_ref[...] += part


def _pool(batch_row3, h1):
    return pl.pallas_call(
        _pool_body,
        grid=(_N // _BN,),
        in_specs=[
            pl.BlockSpec((1, 1, _BN), lambda i: (i, 0, 0)),
            pl.BlockSpec((_BN, _D), lambda i: (i, 0)),
        ],
        out_specs=pl.BlockSpec((_G, _D), lambda i: (0, 0)),
        out_shape=jax.ShapeDtypeStruct((_G, _D), jnp.float32),
    )(batch_row3, h1)


def _head_body(h0_ref, h1_ref, b_ref, hp_ref, a0_ref, a1_ref, a2_ref,
               cb0_ref, cw1_ref, cb1_ref, fw_ref, fb_ref, o_ref):
    oh = (b_ref[0] == lax.broadcasted_iota(jnp.int32, (_BN, _G), 1))
    hp = jnp.dot(oh.astype(jnp.float32), hp_ref[...],
                 preferred_element_type=jnp.float32)
    z = (jnp.dot(h0_ref[...], a0_ref[...], preferred_element_type=jnp.float32)
         + jnp.dot(h1_ref[...], a1_ref[...], preferred_element_type=jnp.float32)
         + jnp.dot(hp, a2_ref[...], preferred_element_type=jnp.float32)
         + cb0_ref[...])
    z = jnp.where(z > 0, z, 0.01 * z)
    z = jnp.dot(z, cw1_ref[...], preferred_element_type=jnp.float32) + cb1_ref[...]
    z = jnp.where(z > 0, z, 0.01 * z)
    y = jnp.dot(z, fw_ref[...], preferred_element_type=jnp.float32) + fb_ref[...]
    o_ref[...] = jax.nn.sigmoid(y)


def _head(h0, h1, batch_col3, hp, a0, a1, a2, cb0, cw1, cb1, fw_pad, fb_pad):
    c1 = cw1.shape[0]
    c2 = cw1.shape[1]
    return pl.pallas_call(
        _head_body,
        grid=(_N // _BN,),
        in_specs=[
            pl.BlockSpec((_BN, _D), lambda i: (i, 0)),
            pl.BlockSpec((_BN, _D), lambda i: (i, 0)),
            pl.BlockSpec((1, _BN, 1), lambda i: (i, 0, 0)),
            pl.BlockSpec((_G, _D), lambda i: (0, 0)),
            pl.BlockSpec((_D, c1), lambda i: (0, 0)),
            pl.BlockSpec((_D, c1), lambda i: (0, 0)),
            pl.BlockSpec((_D, c1), lambda i: (0, 0)),
            pl.BlockSpec((1, c1), lambda i: (0, 0)),
            pl.BlockSpec((c1, c2), lambda i: (0, 0)),
            pl.BlockSpec((1, c2), lambda i: (0, 0)),
            pl.BlockSpec((c2, _D), lambda i: (0, 0)),
            pl.BlockSpec((1, _D), lambda i: (0, 0)),
        ],
        out_specs=pl.BlockSpec((_BN, _D), lambda i: (i, 0)),
        out_shape=jax.ShapeDtypeStruct((_N, _D), jnp.float32),
    )(h0, h1, batch_col3, hp, a0, a1, a2, cb0, cw1, cb1, fw_pad, fb_pad)


def kernel(x, edge_index, edge_attr, batch, params):
    src = edge_index[0]
    dst = edge_index[1]

    w_cat = jnp.concatenate([params['eW0'], params['eW1']], axis=1)
    b_cat = jnp.concatenate([params['eb0'], params['eb1']])[None, :]
    e0, e1 = _edge_transform(edge_attr.T, w_cat, b_cat)

    bn = 1.0 / jnp.sqrt(1.0 + 1e-5)
    s0 = (params['g0'] * bn)[None, :]
    t0 = (params['b0'] * params['g0'] * bn + params['be0'])[None, :]
    s1 = (params['g1'] * bn)[None, :]
    t1 = (params['b1'] * params['g1'] * bn + params['be1'])[None, :]
    eps1_0 = (1.0 + params['eps0']).reshape(1, 1)
    eps1_1 = (1.0 + params['eps1']).reshape(1, 1)

    agg0 = _sc_message(src, dst, e0, x)
    h0 = _node_update(x, agg0, params['W0'], s0, t0, eps1_0)
    agg1 = _sc_message(src, dst, e1, h0)
    h1 = _node_update(h0, agg1, params['W1'], s1, t1, eps1_1)

    batch_row3 = batch.reshape(_N // _BN, 1, _BN)
    batch_col3 = batch.reshape(_N // _BN, _BN, 1)
    hp = _pool(batch_row3, h1)

    a0 = params['cW0'][:_D]
    a1 = params['cW0'][_D:2 * _D]
    a2 = params['cW0'][2 * _D:]
    cb0 = params['cb0'][None, :]
    cb1 = params['cb1'][None, :]
    fw_pad = jnp.concatenate(
        [params['fW'], jnp.zeros((params['fW'].shape[0], _D - 1), jnp.float32)],
        axis=1)
    fb_pad = jnp.concatenate(
        [params['fb'], jnp.zeros((_D - 1,), jnp.float32)])[None, :]

    out_pad = _head(h0, h1, batch_col3, hp, a0, a1, a2, cb0,
                    params['cW1'], cb1, fw_pad, fb_pad)
    return out_pad[:, :1]


# speedup vs baseline: 5.6685x; 1.1611x over previous
import jax
import jax.numpy as jnp
from jax import lax
from jax.experimental import pallas as pl
from jax.experimental.pallas import tpu as pltpu
from jax.experimental.pallas import tpu_sc as plsc

_N = 10000
_E = 320000
_D = 128
_ED = 16
_G = 64

_NC = 2
_NS = 16
_NW = _NC * _NS
_K = 80
_NCHUNKS = _E // _K
_CHUNKS_PER_W = -(-_NCHUNKS // _NW)
_EXPORT_ROWS = 80
_NPIECES = _N // _EXPORT_ROWS
_PIECES_PER_TILE = -(-_NPIECES // _NS)

_E2 = _E // 2
_K2 = _K // 2
_BE = 3200
_BN = 2000
_HIGH = lax.Precision.HIGHEST


def _leaky2(h):
    return jnp.where(h > 0, h, 1e-4 * h)


def _edge_body(ea_a_ref, ea_b_ref, w_ref, b_ref, o0_ref, o1_ref):
    def proj(ea_ref):
        z = lax.dot_general(ea_ref[...].astype(jnp.bfloat16),
                            w_ref[...].astype(jnp.bfloat16),
                            (((0,), (0,)), ((), ())),
                            preferred_element_type=jnp.float32)
        return lax.bitcast_convert_type(z + b_ref[...], jnp.int32)
    za = proj(ea_a_ref)
    zb = proj(ea_b_ref)
    packed = (za & jnp.int32(-65536)) | lax.shift_right_logical(zb, 16)
    o0_ref[...] = packed[:, :_D]
    o1_ref[...] = packed[:, _D:]


def _edge_transform(edge_attr_t, w_cat, b_cat):
    nblk = _E2 // _BE
    return pl.pallas_call(
        _edge_body,
        grid=(nblk,),
        in_specs=[
            pl.BlockSpec((_ED, _BE), lambda i: (0, i)),
            pl.BlockSpec((_ED, _BE), lambda i: (0, i + nblk)),
            pl.BlockSpec((_ED, 2 * _D), lambda i: (0, 0)),
            pl.BlockSpec((1, 2 * _D), lambda i: (0, 0)),
        ],
        out_specs=[
            pl.BlockSpec((_BE, _D), lambda i: (i, 0)),
            pl.BlockSpec((_BE, _D), lambda i: (i, 0)),
        ],
        out_shape=[jax.ShapeDtypeStruct((_E2, _D), jnp.int32)] * 2,
    )(edge_attr_t, edge_attr_t, w_cat, b_cat)


def _sc_message_body(src_hbm, dst_hbm, e_hbm, x_hbm, out_hbm,
                     idx0, idx1, dstv0, dstv1, dsts0, dsts1, xe0, xe1,
                     ev0, ev1, acc_sh,
                     sem_i0, sem_i1, sem_g0, sem_g1, sem_s0, sem_s1):
    c = lax.axis_index("c")
    s = lax.axis_index("s")
    wid = s * _NC + c
    nj = (_NCHUNKS - wid + _NW - 1) // _NW
    slots = ((idx0, dstv0, dsts0, ev0, xe0, sem_i0, sem_g0, sem_s0),
             (idx1, dstv1, dsts1, ev1, xe1, sem_i1, sem_g1, sem_s1))
    xe_v = xe0

    def zrow(r, carry):
        for cc in range(_D // 16):
            xe_v[r, pl.ds(cc * 16, 16)] = jnp.zeros((16,), jnp.float32)
        return carry
    lax.fori_loop(0, _EXPORT_ROWS, zrow, 0)
    for p in range(_PIECES_PER_TILE):
        piece = s + _NS * p

        @pl.when(piece < _NPIECES)
        def _():
            r0 = piece * _EXPORT_ROWS
            pltpu.sync_copy(xe_v.at[pl.ds(0, _EXPORT_ROWS)],
                            acc_sh.at[pl.ds(r0, _EXPORT_ROWS)])
    plsc.subcore_barrier()

    def _prefetch(j, slot, guard):
        idx_b, dst_b, _, e_b, _, sem_i, _, _ = slot

        @pl.when(guard)
        def _():
            chunk = wid + _NW * j
            pltpu.async_copy(src_hbm.at[pl.ds(chunk * _K, _K)], idx_b, sem_i)
            pltpu.async_copy(dst_hbm.at[pl.ds(chunk * _K, _K)], dst_b, sem_i)
            pltpu.async_copy(e_hbm.at[pl.ds(chunk * _K2, _K2)], e_b, sem_i)

    def _wait_prefetch(slot, guard):
        idx_b, dst_b, _, e_b, _, sem_i, _, _ = slot

        @pl.when(guard)
        def _():
            pltpu.make_async_copy(src_hbm.at[pl.ds(0, _K)], idx_b, sem_i).wait()
            pltpu.make_async_copy(dst_hbm.at[pl.ds(0, _K)], dst_b, sem_i).wait()
            pltpu.make_async_copy(e_hbm.at[pl.ds(0, _K)], e_b, sem_i).wait()

    def _wait_scatter(slot, guard):
        _, _, dst2_b, _, xe_b, _, _, sem_s = slot

        @pl.when(guard)
        def _():
            pltpu.make_async_copy(xe_b, acc_sh.at[dst2_b], sem_s).wait()

    def _start_gather(slot, guard):
        idx_b, _, _, _, xe_b, _, sem_g, _ = slot

        @pl.when(guard)
        def _():
            pltpu.async_copy(x_hbm.at[idx_b], xe_b, sem_g)

    def _wait_gather(slot, guard):
        idx_b, _, _, _, xe_b, _, sem_g, _ = slot

        @pl.when(guard)
        def _():
            pltpu.make_async_copy(x_hbm.at[idx_b], xe_b, sem_g).wait()

    def _compute_scatter(slot, guard):
        _, dst_b, dst2_b, e_b, xe_b, _, _, sem_s = slot

        @pl.when(guard)
        def _():
            mask16 = jnp.full((16,), -65536, jnp.int32)
            sh16 = jnp.full((16,), 16, jnp.int32)

            def row(r, rcarry):
                for cc in range(_D // 16):
                    sl = pl.ds(cc * 16, 16)
                    v = e_b[r, sl]
                    ea = lax.bitcast_convert_type(v & mask16, jnp.float32)
                    eb = lax.bitcast_convert_type(lax.shift_left(v, sh16),
                                                  jnp.float32)
                    xe_b[r, sl] = jnp.maximum(xe_b[r, sl] + ea, 0.0)
                    xe_b[r + _K2, sl] = jnp.maximum(xe_b[r + _K2, sl] + eb, 0.0)
                return rcarry
            lax.fori_loop(0, _K2, row, 0)
            for k in range(_K // 16):
                sl = pl.ds(k * 16, 16)
                dst2_b[sl] = dst_b[sl]
            pltpu.async_copy(xe_b, acc_sh.at[dst2_b], sem_s, add=True)

    _prefetch(0, slots[0], 0 < nj)
    _prefetch(1, slots[1], 1 < nj)
    _wait_prefetch(slots[0], 0 < nj)
    _start_gather(slots[0], 0 < nj)

    def pair_body(p, carry):
        for b in range(2):
            j = 2 * p + b
            o = 1 - b
            _wait_prefetch(slots[o], j + 1 < nj)
            _wait_scatter(slots[o], (j >= 1) & (j + 1 < nj))
            _start_gather(slots[o], j + 1 < nj)
            _wait_gather(slots[b], j < nj)
            _compute_scatter(slots[b], j < nj)
            _prefetch(j + 2, slots[b], j + 2 < nj)
        return carry
    lax.fori_loop(0, (_CHUNKS_PER_W + 1) // 2, pair_body, 0)
    _wait_scatter(slots[0], nj >= 1)
    _wait_scatter(slots[1], nj >= 2)
    plsc.subcore_barrier()

    for p in range(_PIECES_PER_TILE):
        piece = s + _NS * p

        @pl.when(piece < _NPIECES)
        def _():
            r0 = piece * _EXPORT_ROWS
            pltpu.sync_copy(acc_sh.at[pl.ds(r0, _EXPORT_ROWS)],
                            xe_v.at[pl.ds(0, _EXPORT_ROWS)])
            pltpu.sync_copy(xe_v.at[pl.ds(0, _EXPORT_ROWS)],
                            out_hbm.at[c, pl.ds(r0, _EXPORT_ROWS)])


def _sc_message(src, dst, e, x):
    mesh = plsc.VectorSubcoreMesh(core_axis_name="c", subcore_axis_name="s")
    f = pl.kernel(
        _sc_message_body,
        out_type=jax.ShapeDtypeStruct((_NC, _N, _D), jnp.float32),
        mesh=mesh,
        scratch_types=(
            [pltpu.VMEM((_K,), jnp.int32)] * 6
            + [pltpu.VMEM((_K, _D), jnp.float32)] * 2
            + [pltpu.VMEM((_K2, _D), jnp.int32)] * 2
            + [pltpu.VMEM_SHARED((_N, _D), jnp.float32)]
            + [pltpu.SemaphoreType.DMA] * 6
        ),
    )
    return f(src, dst, e, x)


def _node_body(inp_ref, agg_ref, w_ref, sc_ref, bi_ref, eps_ref, o_ref):
    z = eps_ref[0, 0] * inp_ref[...] + agg_ref[0] + agg_ref[1]
    h = jnp.dot(z, w_ref[...], preferred_element_type=jnp.float32,
                precision=_HIGH)
    o_ref[...] = _leaky2(h * sc_ref[...] + bi_ref[...])


def _node_update(inp, agg, w, scale, bias, eps1):
    return pl.pallas_call(
        _node_body,
        grid=(_N // _BN,),
        in_specs=[
            pl.BlockSpec((_BN, _D), lambda i: (i, 0)),
            pl.BlockSpec((_NC, _BN, _D), lambda i: (0, i, 0)),
            pl.BlockSpec((_D, _D), lambda i: (0, 0)),
            pl.BlockSpec((1, _D), lambda i: (0, 0)),
            pl.BlockSpec((1, _D), lambda i: (0, 0)),
            pl.BlockSpec(memory_space=pltpu.SMEM),
        ],
        out_specs=pl.BlockSpec((_BN, _D), lambda i: (i, 0)),
        out_shape=jax.ShapeDtypeStruct((_N, _D), jnp.float32),
    )(inp, agg, w, scale, bias, eps1)


def _pool_body(b_ref, h_ref, o_ref):
    i = pl.program_id(0)
    bids = b_ref[0]
    oh = (bids == lax.broadcasted_iota(jnp.int32, (_G, _BN), 0))
    part = jnp.dot(oh.astype(jnp.float32), h_ref[...],
                   preferred_element_type=jnp.float32, precision=_HIGH)

    @pl.when(i == 0)
    def _():
        o_ref[...] = jnp.zeros_like(o_ref)
    o_ref[...] += part


def _pool(batch_row3, h1):
    return pl.pallas_call(
        _pool_body,
        grid=(_N // _BN,),
        in_specs=[
            pl.BlockSpec((1, 1, _BN), lambda i: (i, 0, 0)),
            pl.BlockSpec((_BN, _D), lambda i: (i, 0)),
        ],
        out_specs=pl.BlockSpec((_G, _D), lambda i: (0, 0)),
        out_shape=jax.ShapeDtypeStruct((_G, _D), jnp.float32),
    )(batch_row3, h1)


def _head_body(h0_ref, h1_ref, b_ref, hp_ref, a0_ref, a1_ref, a2_ref,
               cb0_ref, cw1_ref, cb1_ref, fw_ref, fb_ref, o_ref):
    oh = (b_ref[0] == lax.broadcasted_iota(jnp.int32, (_BN, _G), 1))
    hp = jnp.dot(oh.astype(jnp.float32), hp_ref[...],
                 preferred_element_type=jnp.float32)
    z = (jnp.dot(h0_ref[...], a0_ref[...], preferred_element_type=jnp.float32)
         + jnp.dot(h1_ref[...], a1_ref[...], preferred_element_type=jnp.float32)
         + jnp.dot(hp, a2_ref[...], preferred_element_type=jnp.float32)
         + cb0_ref[...])
    z = jnp.where(z > 0, z, 0.01 * z)
    z = jnp.dot(z, cw1_ref[...], preferred_element_type=jnp.float32) + cb1_ref[...]
    z = jnp.where(z > 0, z, 0.01 * z)
    y = jnp.dot(z, fw_ref[...], preferred_element_type=jnp.float32) + fb_ref[...]
    o_ref[...] = jax.nn.sigmoid(y)


def _head(h0, h1, batch_col3, hp, a0, a1, a2, cb0, cw1, cb1, fw_pad, fb_pad):
    c1 = cw1.shape[0]
    c2 = cw1.shape[1]
    return pl.pallas_call(
        _head_body,
        grid=(_N // _BN,),
        in_specs=[
            pl.BlockSpec((_BN, _D), lambda i: (i, 0)),
            pl.BlockSpec((_BN, _D), lambda i: (i, 0)),
            pl.BlockSpec((1, _BN, 1), lambda i: (i, 0, 0)),
            pl.BlockSpec((_G, _D), lambda i: (0, 0)),
            pl.BlockSpec((_D, c1), lambda i: (0, 0)),
            pl.BlockSpec((_D, c1), lambda i: (0, 0)),
            pl.BlockSpec((_D, c1), lambda i: (0, 0)),
            pl.BlockSpec((1, c1), lambda i: (0, 0)),
            pl.BlockSpec((c1, c2), lambda i: (0, 0)),
            pl.BlockSpec((1, c2), lambda i: (0, 0)),
            pl.BlockSpec((c2, _D), lambda i: (0, 0)),
            pl.BlockSpec((1, _D), lambda i: (0, 0)),
        ],
        out_specs=pl.BlockSpec((_BN, _D), lambda i: (i, 0)),
        out_shape=jax.ShapeDtypeStruct((_N, _D), jnp.float32),
    )(h0, h1, batch_col3, hp, a0, a1, a2, cb0, cw1, cb1, fw_pad, fb_pad)


def kernel(x, edge_index, edge_attr, batch, params):
    def _il(v):
        return jnp.stack([v[:_E2].reshape(-1, _K2),
                          v[_E2:].reshape(-1, _K2)], 1).reshape(-1)
    src = _il(edge_index[0])
    dst = _il(edge_index[1])

    w_cat = jnp.concatenate([params['eW0'], params['eW1']], axis=1)
    b_cat = jnp.concatenate([params['eb0'], params['eb1']])[None, :]
    e0, e1 = _edge_transform(edge_attr.T, w_cat, b_cat)

    bn = 1.0 / jnp.sqrt(1.0 + 1e-5)
    s0 = (params['g0'] * bn)[None, :]
    t0 = (params['b0'] * params['g0'] * bn + params['be0'])[None, :]
    s1 = (params['g1'] * bn)[None, :]
    t1 = (params['b1'] * params['g1'] * bn + params['be1'])[None, :]
    eps1_0 = (1.0 + params['eps0']).reshape(1, 1)
    eps1_1 = (1.0 + params['eps1']).reshape(1, 1)

    agg0 = _sc_message(src, dst, e0, x)
    h0 = _node_update(x, agg0, params['W0'], s0, t0, eps1_0)
    agg1 = _sc_message(src, dst, e1, h0)
    h1 = _node_update(h0, agg1, params['W1'], s1, t1, eps1_1)

    batch_row3 = batch.reshape(_N // _BN, 1, _BN)
    batch_col3 = batch.reshape(_N // _BN, _BN, 1)
    hp = _pool(batch_row3, h1)

    a0 = params['cW0'][:_D]
    a1 = params['cW0'][_D:2 * _D]
    a2 = params['cW0'][2 * _D:]
    cb0 = params['cb0'][None, :]
    cb1 = params['cb1'][None, :]
    fw_pad = jnp.concatenate(
        [params['fW'], jnp.zeros((params['fW'].shape[0], _D - 1), jnp.float32)],
        axis=1)
    fb_pad = jnp.concatenate(
        [params['fb'], jnp.zeros((_D - 1,), jnp.float32)])[None, :]

    out_pad = _head(h0, h1, batch_col3, hp, a0, a1, a2, cb0,
                    params['cW1'], cb1, fw_pad, fb_pad)
    return out_pad[:, :1]


# speedup vs baseline: 6.3836x; 1.1262x over previous
import jax
import jax.numpy as jnp
from jax import lax
from jax.experimental import pallas as pl
from jax.experimental.pallas import tpu as pltpu
from jax.experimental.pallas import tpu_sc as plsc

_N = 10000
_E = 320000
_D = 128
_ED = 16
_G = 64

_NC = 2
_NS = 16
_NW = _NC * _NS
_K = 80
_NCHUNKS = _E // _K
_CHUNKS_PER_W = -(-_NCHUNKS // _NW)
_EXPORT_ROWS = 80
_NPIECES = _N // _EXPORT_ROWS
_PIECES_PER_TILE = -(-_NPIECES // _NS)

_E2 = _E // 2
_K2 = _K // 2
_BE = 3200
_BN = 2000
_HIGH = lax.Precision.HIGHEST


def _leaky2(h):
    return jnp.where(h > 0, h, 1e-4 * h)


def _edge_body(ea_a_ref, ea_b_ref, w_ref, b_ref, o0_ref, o1_ref):
    def proj(ea_ref):
        z = lax.dot_general(ea_ref[...].astype(jnp.bfloat16),
                            w_ref[...].astype(jnp.bfloat16),
                            (((0,), (0,)), ((), ())),
                            preferred_element_type=jnp.float32)
        return lax.bitcast_convert_type(z + b_ref[...], jnp.int32)
    za = proj(ea_a_ref)
    zb = proj(ea_b_ref)
    packed = (za & jnp.int32(-65536)) | lax.shift_right_logical(zb, 16)
    o0_ref[...] = packed[:, :_D]
    o1_ref[...] = packed[:, _D:]


def _edge_transform(edge_attr_t, w_cat, b_cat):
    nblk = _E2 // _BE
    return pl.pallas_call(
        _edge_body,
        grid=(nblk,),
        in_specs=[
            pl.BlockSpec((_ED, _BE), lambda i: (0, i)),
            pl.BlockSpec((_ED, _BE), lambda i: (0, i + nblk)),
            pl.BlockSpec((_ED, 2 * _D), lambda i: (0, 0)),
            pl.BlockSpec((1, 2 * _D), lambda i: (0, 0)),
        ],
        out_specs=[
            pl.BlockSpec((_BE, _D), lambda i: (i, 0)),
            pl.BlockSpec((_BE, _D), lambda i: (i, 0)),
        ],
        out_shape=[jax.ShapeDtypeStruct((_E2, _D), jnp.int32)] * 2,
    )(edge_attr_t, edge_attr_t, w_cat, b_cat)


def _sc_message_body(src_hbm, dst_hbm, e_hbm, x_hbm, out_hbm,
                     idx0, idx1, dstv0, dstv1, dsts0, dsts1, xe0, xe1,
                     ev0, ev1, acc_sh,
                     sem_i0, sem_i1, sem_g0, sem_g1, sem_s0, sem_s1):
    c = lax.axis_index("c")
    s = lax.axis_index("s")
    wid = s * _NC + c
    nj = (_NCHUNKS - wid + _NW - 1) // _NW
    slots = ((idx0, dstv0, dsts0, ev0, xe0, sem_i0, sem_g0, sem_s0),
             (idx1, dstv1, dsts1, ev1, xe1, sem_i1, sem_g1, sem_s1))
    xe_v = xe0

    def zrow(r, carry):
        for cc in range(_D // 16):
            xe_v[r, pl.ds(cc * 16, 16)] = jnp.zeros((16,), jnp.float32)
        return carry
    lax.fori_loop(0, _EXPORT_ROWS, zrow, 0)
    for p in range(_PIECES_PER_TILE):
        piece = s + _NS * p

        @pl.when(piece < _NPIECES)
        def _():
            r0 = piece * _EXPORT_ROWS
            pltpu.sync_copy(xe_v.at[pl.ds(0, _EXPORT_ROWS)],
                            acc_sh.at[pl.ds(r0, _EXPORT_ROWS)])
    plsc.subcore_barrier()

    def _prefetch(j, slot, guard):
        idx_b, dst_b, _, e_b, _, sem_i, _, _ = slot

        @pl.when(guard)
        def _():
            chunk = wid + _NW * j
            base = chunk * _K2
            pltpu.async_copy(src_hbm.at[pl.ds(base, _K2)],
                             idx_b.at[pl.ds(0, _K2)], sem_i)
            pltpu.async_copy(src_hbm.at[pl.ds(_E2 + base, _K2)],
                             idx_b.at[pl.ds(_K2, _K2)], sem_i)
            pltpu.async_copy(dst_hbm.at[pl.ds(base, _K2)],
                             dst_b.at[pl.ds(0, _K2)], sem_i)
            pltpu.async_copy(dst_hbm.at[pl.ds(_E2 + base, _K2)],
                             dst_b.at[pl.ds(_K2, _K2)], sem_i)
            pltpu.async_copy(e_hbm.at[pl.ds(base, _K2)], e_b, sem_i)

    def _wait_prefetch(slot, guard):
        idx_b, dst_b, _, e_b, _, sem_i, _, _ = slot

        @pl.when(guard)
        def _():
            for half in range(2):
                pltpu.make_async_copy(src_hbm.at[pl.ds(0, _K2)],
                                      idx_b.at[pl.ds(half * _K2, _K2)],
                                      sem_i).wait()
                pltpu.make_async_copy(dst_hbm.at[pl.ds(0, _K2)],
                                      dst_b.at[pl.ds(half * _K2, _K2)],
                                      sem_i).wait()
            pltpu.make_async_copy(e_hbm.at[pl.ds(0, _K2)], e_b, sem_i).wait()

    def _wait_scatter(slot, guard):
        _, _, dst2_b, _, xe_b, _, _, sem_s = slot

        @pl.when(guard)
        def _():
            pltpu.make_async_copy(xe_b, acc_sh.at[dst2_b], sem_s).wait()

    def _start_gather(slot, guard):
        idx_b, _, _, _, xe_b, _, sem_g, _ = slot

        @pl.when(guard)
        def _():
            pltpu.async_copy(x_hbm.at[idx_b], xe_b, sem_g)

    def _wait_gather(slot, guard):
        idx_b, _, _, _, xe_b, _, sem_g, _ = slot

        @pl.when(guard)
        def _():
            pltpu.make_async_copy(x_hbm.at[idx_b], xe_b, sem_g).wait()

    def _compute_scatter(slot, guard):
        _, dst_b, dst2_b, e_b, xe_b, _, _, sem_s = slot

        @pl.when(guard)
        def _():
            mask16 = jnp.full((16,), -65536, jnp.int32)
            sh16 = jnp.full((16,), 16, jnp.int32)

            def row(r, rcarry):
                for cc in range(_D // 16):
                    sl = pl.ds(cc * 16, 16)
                    v = e_b[r, sl]
                    ea = lax.bitcast_convert_type(v & mask16, jnp.float32)
                    eb = lax.bitcast_convert_type(lax.shift_left(v, sh16),
                                                  jnp.float32)
                    xe_b[r, sl] = jnp.maximum(xe_b[r, sl] + ea, 0.0)
                    xe_b[r + _K2, sl] = jnp.maximum(xe_b[r + _K2, sl] + eb, 0.0)
                return rcarry
            lax.fori_loop(0, _K2, row, 0)
            for k in range(_K // 16):
                sl = pl.ds(k * 16, 16)
                dst2_b[sl] = dst_b[sl]
            pltpu.async_copy(xe_b, acc_sh.at[dst2_b], sem_s, add=True)

    _prefetch(0, slots[0], 0 < nj)
    _prefetch(1, slots[1], 1 < nj)
    _wait_prefetch(slots[0], 0 < nj)
    _start_gather(slots[0], 0 < nj)

    def pair_body(p, carry):
        for b in range(2):
            j = 2 * p + b
            o = 1 - b
            _wait_prefetch(slots[o], j + 1 < nj)
            _wait_scatter(slots[o], (j >= 1) & (j + 1 < nj))
            _start_gather(slots[o], j + 1 < nj)
            _wait_gather(slots[b], j < nj)
            _compute_scatter(slots[b], j < nj)
            _prefetch(j + 2, slots[b], j + 2 < nj)
        return carry
    lax.fori_loop(0, (_CHUNKS_PER_W + 1) // 2, pair_body, 0)
    _wait_scatter(slots[0], nj >= 1)
    _wait_scatter(slots[1], nj >= 2)
    plsc.subcore_barrier()

    for p in range(_PIECES_PER_TILE):
        piece = s + _NS * p

        @pl.when(piece < _NPIECES)
        def _():
            r0 = piece * _EXPORT_ROWS
            pltpu.sync_copy(acc_sh.at[pl.ds(r0, _EXPORT_ROWS)],
                            xe_v.at[pl.ds(0, _EXPORT_ROWS)])
            pltpu.sync_copy(xe_v.at[pl.ds(0, _EXPORT_ROWS)],
                            out_hbm.at[c, pl.ds(r0, _EXPORT_ROWS)])


def _sc_message(src, dst, e, x):
    mesh = plsc.VectorSubcoreMesh(core_axis_name="c", subcore_axis_name="s")
    f = pl.kernel(
        _sc_message_body,
        out_type=jax.ShapeDtypeStruct((_NC, _N, _D), jnp.float32),
        mesh=mesh,
        scratch_types=(
            [pltpu.VMEM((_K,), jnp.int32)] * 6
            + [pltpu.VMEM((_K, _D), jnp.float32)] * 2
            + [pltpu.VMEM((_K2, _D), jnp.int32)] * 2
            + [pltpu.VMEM_SHARED((_N, _D), jnp.float32)]
            + [pltpu.SemaphoreType.DMA] * 6
        ),
    )
    return f(src, dst, e, x)


def _node_body(inp_ref, agg_ref, w_ref, sc_ref, bi_ref, eps_ref, o_ref):
    z = eps_ref[0, 0] * inp_ref[...] + agg_ref[0] + agg_ref[1]
    h = jnp.dot(z, w_ref[...], preferred_element_type=jnp.float32,
                precision=_HIGH)
    o_ref[...] = _leaky2(h * sc_ref[...] + bi_ref[...])


def _node_update(inp, agg, w, scale, bias, eps1):
    return pl.pallas_call(
        _node_body,
        grid=(_N // _BN,),
        in_specs=[
            pl.BlockSpec((_BN, _D), lambda i: (i, 0)),
            pl.BlockSpec((_NC, _BN, _D), lambda i: (0, i, 0)),
            pl.BlockSpec((_D, _D), lambda i: (0, 0)),
            pl.BlockSpec((1, _D), lambda i: (0, 0)),
            pl.BlockSpec((1, _D), lambda i: (0, 0)),
            pl.BlockSpec(memory_space=pltpu.SMEM),
        ],
        out_specs=pl.BlockSpec((_BN, _D), lambda i: (i, 0)),
        out_shape=jax.ShapeDtypeStruct((_N, _D), jnp.float32),
    )(inp, agg, w, scale, bias, eps1)


def _pool_body(b_ref, h_ref, o_ref):
    i = pl.program_id(0)
    bids = b_ref[0]
    oh = (bids == lax.broadcasted_iota(jnp.int32, (_G, _BN), 0))
    part = jnp.dot(oh.astype(jnp.float32), h_ref[...],
                   preferred_element_type=jnp.float32, precision=_HIGH)

    @pl.when(i == 0)
    def _():
        o_ref[...] = jnp.zeros_like(o_ref)
    o_ref[...] += part


def _pool(batch_row3, h1):
    return pl.pallas_call(
        _pool_body,
        grid=(_N // _BN,),
        in_specs=[
            pl.BlockSpec((1, 1, _BN), lambda i: (i, 0, 0)),
            pl.BlockSpec((_BN, _D), lambda i: (i, 0)),
        ],
        out_specs=pl.BlockSpec((_G, _D), lambda i: (0, 0)),
        out_shape=jax.ShapeDtypeStruct((_G, _D), jnp.float32),
    )(batch_row3, h1)


def _head_body(h0_ref, h1_ref, b_ref, hp_ref, a0_ref, a1_ref, a2_ref,
               cb0_ref, cw1_ref, cb1_ref, fw_ref, fb_ref, o_ref):
    oh_t = (b_ref[0] == lax.broadcasted_iota(jnp.int32, (_G, _BN), 0))
    hp = lax.dot_general(oh_t.astype(jnp.float32), hp_ref[...],
                         (((0,), (0,)), ((), ())),
                         preferred_element_type=jnp.float32)
    z = (jnp.dot(h0_ref[...], a0_ref[...], preferred_element_type=jnp.float32)
         + jnp.dot(h1_ref[...], a1_ref[...], preferred_element_type=jnp.float32)
         + jnp.dot(hp, a2_ref[...], preferred_element_type=jnp.float32)
         + cb0_ref[...])
    z = jnp.where(z > 0, z, 0.01 * z)
    z = jnp.dot(z, cw1_ref[...], preferred_element_type=jnp.float32) + cb1_ref[...]
    z = jnp.where(z > 0, z, 0.01 * z)
    y = jnp.dot(z, fw_ref[...], preferred_element_type=jnp.float32) + fb_ref[...]
    o_ref[...] = jax.nn.sigmoid(y)


def _head(h0, h1, batch_col3, hp, a0, a1, a2, cb0, cw1, cb1, fw_pad, fb_pad):
    c1 = cw1.shape[0]
    c2 = cw1.shape[1]
    return pl.pallas_call(
        _head_body,
        grid=(_N // _BN,),
        in_specs=[
            pl.BlockSpec((_BN, _D), lambda i: (i, 0)),
            pl.BlockSpec((_BN, _D), lambda i: (i, 0)),
            pl.BlockSpec((1, 1, _BN), lambda i: (i, 0, 0)),
            pl.BlockSpec((_G, _D), lambda i: (0, 0)),
            pl.BlockSpec((_D, c1), lambda i: (0, 0)),
            pl.BlockSpec((_D, c1), lambda i: (0, 0)),
            pl.BlockSpec((_D, c1), lambda i: (0, 0)),
            pl.BlockSpec((1, c1), lambda i: (0, 0)),
            pl.BlockSpec((c1, c2), lambda i: (0, 0)),
            pl.BlockSpec((1, c2), lambda i: (0, 0)),
            pl.BlockSpec((c2, _D), lambda i: (0, 0)),
            pl.BlockSpec((1, _D), lambda i: (0, 0)),
        ],
        out_specs=pl.BlockSpec((_BN, _D), lambda i: (i, 0)),
        out_shape=jax.ShapeDtypeStruct((_N, _D), jnp.float32),
    )(h0, h1, batch_col3, hp, a0, a1, a2, cb0, cw1, cb1, fw_pad, fb_pad)


def kernel(x, edge_index, edge_attr, batch, params):
    src = edge_index[0]
    dst = edge_index[1]

    w_cat = jnp.concatenate([params['eW0'], params['eW1']], axis=1)
    b_cat = jnp.concatenate([params['eb0'], params['eb1']])[None, :]
    e0, e1 = _edge_transform(edge_attr.T, w_cat, b_cat)

    bn = 1.0 / jnp.sqrt(1.0 + 1e-5)
    s0 = (params['g0'] * bn)[None, :]
    t0 = (params['b0'] * params['g0'] * bn + params['be0'])[None, :]
    s1 = (params['g1'] * bn)[None, :]
    t1 = (params['b1'] * params['g1'] * bn + params['be1'])[None, :]
    eps1_0 = (1.0 + params['eps0']).reshape(1, 1)
    eps1_1 = (1.0 + params['eps1']).reshape(1, 1)

    agg0 = _sc_message(src, dst, e0, x)
    h0 = _node_update(x, agg0, params['W0'], s0, t0, eps1_0)
    agg1 = _sc_message(src, dst, e1, h0)
    h1 = _node_update(h0, agg1, params['W1'], s1, t1, eps1_1)

    batch_row3 = batch.reshape(_N // _BN, 1, _BN)
    hp = _pool(batch_row3, h1)

    a0 = params['cW0'][:_D]
    a1 = params['cW0'][_D:2 * _D]
    a2 = params['cW0'][2 * _D:]
    cb0 = params['cb0'][None, :]
    cb1 = params['cb1'][None, :]
    fw_pad = jnp.concatenate(
        [params['fW'], jnp.zeros((params['fW'].shape[0], _D - 1), jnp.float32)],
        axis=1)
    fb_pad = jnp.concatenate(
        [params['fb'], jnp.zeros((_D - 1,), jnp.float32)])[None, :]

    out_pad = _head(h0, h1, batch_row3, hp, a0, a1, a2, cb0,
                    params['cW1'], cb1, fw_pad, fb_pad)
    return out_pad[:, :1]


# speedup vs baseline: 6.6400x; 1.0402x over previous
import jax
import jax.numpy as jnp
from jax import lax
from jax.experimental import pallas as pl
from jax.experimental.pallas import tpu as pltpu
from jax.experimental.pallas import tpu_sc as plsc

_N = 10000
_E = 320000
_D = 128
_ED = 16
_G = 64

_NC = 2
_NS = 16
_NW = _NC * _NS
_K = 80
_NCHUNKS = _E // _K
_CHUNKS_PER_W = -(-_NCHUNKS // _NW)
_EXPORT_ROWS = 80
_NPIECES = _N // _EXPORT_ROWS
_PIECES_PER_TILE = -(-_NPIECES // _NS)

_E2 = _E // 2
_K2 = _K // 2
_BE = 6400
_BN = 2000
_HIGH = lax.Precision.HIGHEST


def _leaky2(h):
    return jnp.where(h > 0, h, 1e-4 * h)


def _edge_body(ea_a_ref, ea_b_ref, w_ref, b_ref, o_ref):
    def proj(ea_ref):
        z = lax.dot_general(ea_ref[...].astype(jnp.bfloat16),
                            w_ref[...].astype(jnp.bfloat16),
                            (((0,), (0,)), ((), ())),
                            preferred_element_type=jnp.float32)
        return lax.bitcast_convert_type(z + b_ref[...], jnp.int32)
    za = proj(ea_a_ref)
    zb = proj(ea_b_ref)
    o_ref[...] = (za & jnp.int32(-65536)) | lax.shift_right_logical(zb, 16)


def _edge_transform(edge_attr_t, w, b):
    nblk = _E2 // _BE
    return pl.pallas_call(
        _edge_body,
        grid=(nblk,),
        in_specs=[
            pl.BlockSpec((_ED, _BE), lambda i: (0, i)),
            pl.BlockSpec((_ED, _BE), lambda i: (0, i + nblk)),
            pl.BlockSpec((_ED, _D), lambda i: (0, 0)),
            pl.BlockSpec((1, _D), lambda i: (0, 0)),
        ],
        out_specs=pl.BlockSpec((_BE, _D), lambda i: (i, 0)),
        out_shape=jax.ShapeDtypeStruct((_E2, _D), jnp.int32),
    )(edge_attr_t, edge_attr_t, w, b)


def _sc_message_body(src_hbm, dst_hbm, e_hbm, x_hbm, out_hbm,
                     idx0, idx1, dstv0, dstv1, dsts0, dsts1, xe0, xe1,
                     ev0, ev1, acc_sh,
                     sem_i0, sem_i1, sem_g0, sem_g1, sem_s0, sem_s1):
    c = lax.axis_index("c")
    s = lax.axis_index("s")
    wid = s * _NC + c
    nj = (_NCHUNKS - wid + _NW - 1) // _NW
    slots = ((idx0, dstv0, dsts0, ev0, xe0, sem_i0, sem_g0, sem_s0),
             (idx1, dstv1, dsts1, ev1, xe1, sem_i1, sem_g1, sem_s1))
    xe_v = xe0

    def zrow(r, carry):
        for cc in range(_D // 16):
            xe_v[r, pl.ds(cc * 16, 16)] = jnp.zeros((16,), jnp.float32)
        return carry
    lax.fori_loop(0, _EXPORT_ROWS, zrow, 0)
    for p in range(_PIECES_PER_TILE):
        piece = s + _NS * p

        @pl.when(piece < _NPIECES)
        def _():
            r0 = piece * _EXPORT_ROWS
            pltpu.sync_copy(xe_v.at[pl.ds(0, _EXPORT_ROWS)],
                            acc_sh.at[pl.ds(r0, _EXPORT_ROWS)])
    plsc.subcore_barrier()

    def _prefetch(j, slot, guard):
        idx_b, dst_b, _, e_b, _, sem_i, _, _ = slot

        @pl.when(guard)
        def _():
            chunk = wid + _NW * j
            base = chunk * _K2
            pltpu.async_copy(src_hbm.at[pl.ds(base, _K2)],
                             idx_b.at[pl.ds(0, _K2)], sem_i)
            pltpu.async_copy(src_hbm.at[pl.ds(_E2 + base, _K2)],
                             idx_b.at[pl.ds(_K2, _K2)], sem_i)
            pltpu.async_copy(dst_hbm.at[pl.ds(base, _K2)],
                             dst_b.at[pl.ds(0, _K2)], sem_i)
            pltpu.async_copy(dst_hbm.at[pl.ds(_E2 + base, _K2)],
                             dst_b.at[pl.ds(_K2, _K2)], sem_i)
            pltpu.async_copy(e_hbm.at[pl.ds(base, _K2)], e_b, sem_i)

    def _wait_prefetch(slot, guard):
        idx_b, dst_b, _, e_b, _, sem_i, _, _ = slot

        @pl.when(guard)
        def _():
            for half in range(2):
                pltpu.make_async_copy(src_hbm.at[pl.ds(0, _K2)],
                                      idx_b.at[pl.ds(half * _K2, _K2)],
                                      sem_i).wait()
                pltpu.make_async_copy(dst_hbm.at[pl.ds(0, _K2)],
                                      dst_b.at[pl.ds(half * _K2, _K2)],
                                      sem_i).wait()
            pltpu.make_async_copy(e_hbm.at[pl.ds(0, _K2)], e_b, sem_i).wait()

    def _wait_scatter(slot, guard):
        _, _, dst2_b, _, xe_b, _, _, sem_s = slot

        @pl.when(guard)
        def _():
            pltpu.make_async_copy(xe_b, acc_sh.at[dst2_b], sem_s).wait()

    def _start_gather(slot, guard):
        idx_b, _, _, _, xe_b, _, sem_g, _ = slot

        @pl.when(guard)
        def _():
            pltpu.async_copy(x_hbm.at[idx_b], xe_b, sem_g)

    def _wait_gather(slot, guard):
        idx_b, _, _, _, xe_b, _, sem_g, _ = slot

        @pl.when(guard)
        def _():
            pltpu.make_async_copy(x_hbm.at[idx_b], xe_b, sem_g).wait()

    def _compute_scatter(slot, guard):
        _, dst_b, dst2_b, e_b, xe_b, _, _, sem_s = slot

        @pl.when(guard)
        def _():
            mask16 = jnp.full((16,), -65536, jnp.int32)
            sh16 = jnp.full((16,), 16, jnp.int32)

            def row(r, rcarry):
                for cc in range(_D // 16):
                    sl = pl.ds(cc * 16, 16)
                    v = e_b[r, sl]
                    ea = lax.bitcast_convert_type(v & mask16, jnp.float32)
                    eb = lax.bitcast_convert_type(lax.shift_left(v, sh16),
                                                  jnp.float32)
                    xe_b[r, sl] = jnp.maximum(xe_b[r, sl] + ea, 0.0)
                    xe_b[r + _K2, sl] = jnp.maximum(xe_b[r + _K2, sl] + eb, 0.0)
                return rcarry
            lax.fori_loop(0, _K2, row, 0)
            for k in range(_K // 16):
                sl = pl.ds(k * 16, 16)
                dst2_b[sl] = dst_b[sl]
            pltpu.async_copy(xe_b, acc_sh.at[dst2_b], sem_s, add=True)

    _prefetch(0, slots[0], 0 < nj)
    _prefetch(1, slots[1], 1 < nj)
    _wait_prefetch(slots[0], 0 < nj)
    _start_gather(slots[0], 0 < nj)

    def pair_body(p, carry):
        for b in range(2):
            j = 2 * p + b
            o = 1 - b
            _wait_prefetch(slots[o], j + 1 < nj)
            _wait_scatter(slots[o], (j >= 1) & (j + 1 < nj))
            _start_gather(slots[o], j + 1 < nj)
            _wait_gather(slots[b], j < nj)
            _compute_scatter(slots[b], j < nj)
            _prefetch(j + 2, slots[b], j + 2 < nj)
        return carry
    lax.fori_loop(0, (_CHUNKS_PER_W + 1) // 2, pair_body, 0)
    _wait_scatter(slots[0], nj >= 1)
    _wait_scatter(slots[1], nj >= 2)
    plsc.subcore_barrier()

    for p in range(_PIECES_PER_TILE):
        piece = s + _NS * p

        @pl.when(piece < _NPIECES)
        def _():
            r0 = piece * _EXPORT_ROWS
            pltpu.sync_copy(acc_sh.at[pl.ds(r0, _EXPORT_ROWS)],
                            xe_v.at[pl.ds(0, _EXPORT_ROWS)])
            pltpu.sync_copy(xe_v.at[pl.ds(0, _EXPORT_ROWS)],
                            out_hbm.at[c, pl.ds(r0, _EXPORT_ROWS)])


def _sc_message(src, dst, e, x):
    mesh = plsc.VectorSubcoreMesh(core_axis_name="c", subcore_axis_name="s")
    f = pl.kernel(
        _sc_message_body,
        out_type=jax.ShapeDtypeStruct((_NC, _N, _D), jnp.float32),
        mesh=mesh,
        scratch_types=(
            [pltpu.VMEM((_K,), jnp.int32)] * 6
            + [pltpu.VMEM((_K, _D), jnp.float32)] * 2
            + [pltpu.VMEM((_K2, _D), jnp.int32)] * 2
            + [pltpu.VMEM_SHARED((_N, _D), jnp.float32)]
            + [pltpu.SemaphoreType.DMA] * 6
        ),
    )
    return f(src, dst, e, x)


def _node_body(inp_ref, agg_ref, w_ref, sc_ref, bi_ref, eps_ref, o_ref):
    z = eps_ref[0, 0] * inp_ref[...] + agg_ref[0] + agg_ref[1]
    h = jnp.dot(z, w_ref[...], preferred_element_type=jnp.float32,
                precision=_HIGH)
    o_ref[...] = _leaky2(h * sc_ref[...] + bi_ref[...])


def _node_update(inp, agg, w, scale, bias, eps1):
    return pl.pallas_call(
        _node_body,
        grid=(_N // _BN,),
        in_specs=[
            pl.BlockSpec((_BN, _D), lambda i: (i, 0)),
            pl.BlockSpec((_NC, _BN, _D), lambda i: (0, i, 0)),
            pl.BlockSpec((_D, _D), lambda i: (0, 0)),
            pl.BlockSpec((1, _D), lambda i: (0, 0)),
            pl.BlockSpec((1, _D), lambda i: (0, 0)),
            pl.BlockSpec(memory_space=pltpu.SMEM),
        ],
        out_specs=pl.BlockSpec((_BN, _D), lambda i: (i, 0)),
        out_shape=jax.ShapeDtypeStruct((_N, _D), jnp.float32),
    )(inp, agg, w, scale, bias, eps1)


def _pool_body(b_ref, h_ref, o_ref):
    i = pl.program_id(0)
    bids = b_ref[0]
    oh = (bids == lax.broadcasted_iota(jnp.int32, (_G, _BN), 0))
    part = jnp.dot(oh.astype(jnp.float32), h_ref[...],
                   preferred_element_type=jnp.float32, precision=_HIGH)

    @pl.when(i == 0)
    def _():
        o_ref[...] = jnp.zeros_like(o_ref)
    o_ref[...] += part


def _pool(batch_row3, h1):
    return pl.pallas_call(
        _pool_body,
        grid=(_N // _BN,),
        in_specs=[
            pl.BlockSpec((1, 1, _BN), lambda i: (i, 0, 0)),
            pl.BlockSpec((_BN, _D), lambda i: (i, 0)),
        ],
        out_specs=pl.BlockSpec((_G, _D), lambda i: (0, 0)),
        out_shape=jax.ShapeDtypeStruct((_G, _D), jnp.float32),
    )(batch_row3, h1)


def _head_body(h0_ref, h1_ref, b_ref, hp_ref, a0_ref, a1_ref, a2_ref,
               cb0_ref, cw1_ref, cb1_ref, fw_ref, fb_ref, o_ref):
    oh_t = (b_ref[0] == lax.broadcasted_iota(jnp.int32, (_G, _BN), 0))
    hp = lax.dot_general(oh_t.astype(jnp.float32), hp_ref[...],
                         (((0,), (0,)), ((), ())),
                         preferred_element_type=jnp.float32)
    z = (jnp.dot(h0_ref[...], a0_ref[...], preferred_element_type=jnp.float32)
         + jnp.dot(h1_ref[...], a1_ref[...], preferred_element_type=jnp.float32)
         + jnp.dot(hp, a2_ref[...], preferred_element_type=jnp.float32)
         + cb0_ref[...])
    z = jnp.where(z > 0, z, 0.01 * z)
    z = jnp.dot(z, cw1_ref[...], preferred_element_type=jnp.float32) + cb1_ref[...]
    z = jnp.where(z > 0, z, 0.01 * z)
    y = jnp.dot(z, fw_ref[...], preferred_element_type=jnp.float32) + fb_ref[...]
    o_ref[...] = jax.nn.sigmoid(y)


def _head(h0, h1, batch_col3, hp, a0, a1, a2, cb0, cw1, cb1, fw_pad, fb_pad):
    c1 = cw1.shape[0]
    c2 = cw1.shape[1]
    return pl.pallas_call(
        _head_body,
        grid=(_N // _BN,),
        in_specs=[
            pl.BlockSpec((_BN, _D), lambda i: (i, 0)),
            pl.BlockSpec((_BN, _D), lambda i: (i, 0)),
            pl.BlockSpec((1, 1, _BN), lambda i: (i, 0, 0)),
            pl.BlockSpec((_G, _D), lambda i: (0, 0)),
            pl.BlockSpec((_D, c1), lambda i: (0, 0)),
            pl.BlockSpec((_D, c1), lambda i: (0, 0)),
            pl.BlockSpec((_D, c1), lambda i: (0, 0)),
            pl.BlockSpec((1, c1), lambda i: (0, 0)),
            pl.BlockSpec((c1, c2), lambda i: (0, 0)),
            pl.BlockSpec((1, c2), lambda i: (0, 0)),
            pl.BlockSpec((c2, _D), lambda i: (0, 0)),
            pl.BlockSpec((1, _D), lambda i: (0, 0)),
        ],
        out_specs=pl.BlockSpec((_BN, _D), lambda i: (i, 0)),
        out_shape=jax.ShapeDtypeStruct((_N, _D), jnp.float32),
    )(h0, h1, batch_col3, hp, a0, a1, a2, cb0, cw1, cb1, fw_pad, fb_pad)


def kernel(x, edge_index, edge_attr, batch, params):
    src = edge_index[0]
    dst = edge_index[1]

    ea_t = edge_attr.T
    e0 = _edge_transform(ea_t, params['eW0'], params['eb0'][None, :])
    e1 = _edge_transform(ea_t, params['eW1'], params['eb1'][None, :])

    bn = 1.0 / jnp.sqrt(1.0 + 1e-5)
    s0 = (params['g0'] * bn)[None, :]
    t0 = (params['b0'] * params['g0'] * bn + params['be0'])[None, :]
    s1 = (params['g1'] * bn)[None, :]
    t1 = (params['b1'] * params['g1'] * bn + params['be1'])[None, :]
    eps1_0 = (1.0 + params['eps0']).reshape(1, 1)
    eps1_1 = (1.0 + params['eps1']).reshape(1, 1)

    agg0 = _sc_message(src, dst, e0, x)
    h0 = _node_update(x, agg0, params['W0'], s0, t0, eps1_0)
    agg1 = _sc_message(src, dst, e1, h0)
    h1 = _node_update(h0, agg1, params['W1'], s1, t1, eps1_1)

    batch_row3 = batch.reshape(_N // _BN, 1, _BN)
    hp = _pool(batch_row3, h1)

    a0 = params['cW0'][:_D]
    a1 = params['cW0'][_D:2 * _D]
    a2 = params['cW0'][2 * _D:]
    cb0 = params['cb0'][None, :]
    cb1 = params['cb1'][None, :]
    fw_pad = jnp.concatenate(
        [params['fW'], jnp.zeros((params['fW'].shape[0], _D - 1), jnp.float32)],
        axis=1)
    fb_pad = jnp.concatenate(
        [params['fb'], jnp.zeros((_D - 1,), jnp.float32)])[None, :]

    out_pad = _head(h0, h1, batch_row3, hp, a0, a1, a2, cb0,
                    params['cW1'], cb1, fw_pad, fb_pad)
    return out_pad[:, :1]


# speedup vs baseline: 6.7180x; 1.0117x over previous
import jax
import jax.numpy as jnp
from jax import lax
from jax.experimental import pallas as pl
from jax.experimental.pallas import tpu as pltpu
from jax.experimental.pallas import tpu_sc as plsc

_N = 10000
_E = 320000
_D = 128
_ED = 16
_G = 64

_NC = 2
_NS = 16
_NW = _NC * _NS
_K = 80
_NCHUNKS = _E // _K
_CHUNKS_PER_W = -(-_NCHUNKS // _NW)
_EXPORT_ROWS = 80
_NPIECES = _N // _EXPORT_ROWS
_PIECES_PER_TILE = -(-_NPIECES // _NS)

_E2 = _E // 2
_K2 = _K // 2
_BE = 6400
_BN = 2000
_HIGH = lax.Precision.HIGHEST


def _leaky2(h):
    return jnp.where(h > 0, h, 1e-4 * h)


def _edge_body(ea_a_ref, ea_b_ref, w_ref, b_ref, o_ref):
    def proj(ea_ref):
        z = lax.dot_general(ea_ref[...].astype(jnp.bfloat16),
                            w_ref[...].astype(jnp.bfloat16),
                            (((0,), (0,)), ((), ())),
                            preferred_element_type=jnp.float32)
        return lax.bitcast_convert_type(z + b_ref[...], jnp.int32)
    za = proj(ea_a_ref)
    zb = proj(ea_b_ref)
    o_ref[...] = (za & jnp.int32(-65536)) | lax.shift_right_logical(zb, 16)


def _edge_transform(edge_attr_t, w, b):
    nblk = _E2 // _BE
    return pl.pallas_call(
        _edge_body,
        grid=(nblk,),
        in_specs=[
            pl.BlockSpec((_ED, _BE), lambda i: (0, i)),
            pl.BlockSpec((_ED, _BE), lambda i: (0, i + nblk)),
            pl.BlockSpec((_ED, _D), lambda i: (0, 0)),
            pl.BlockSpec((1, _D), lambda i: (0, 0)),
        ],
        out_specs=pl.BlockSpec((_BE, _D), lambda i: (i, 0)),
        out_shape=jax.ShapeDtypeStruct((_E2, _D), jnp.int32),
    )(edge_attr_t, edge_attr_t, w, b)


def _sc_message_body(src_hbm, dst_hbm, e_hbm, x_hbm, out_hbm,
                     idx0, idx1, dstv0, dstv1, dsts0, dsts1, xe0, xe1,
                     ev0, ev1, acc_sh,
                     sem_i0, sem_i1, sem_g0, sem_g1, sem_s0, sem_s1):
    c = lax.axis_index("c")
    s = lax.axis_index("s")
    wid = s * _NC + c
    nj = (_NCHUNKS - wid + _NW - 1) // _NW
    slots = ((idx0, dstv0, dsts0, ev0, xe0, sem_i0, sem_g0, sem_s0),
             (idx1, dstv1, dsts1, ev1, xe1, sem_i1, sem_g1, sem_s1))
    xe_v = xe0

    def _prefetch(j, slot, guard):
        idx_b, dst_b, _, e_b, _, sem_i, _, _ = slot

        @pl.when(guard)
        def _():
            chunk = wid + _NW * j
            base = chunk * _K2
            pltpu.async_copy(src_hbm.at[pl.ds(base, _K2)],
                             idx_b.at[pl.ds(0, _K2)], sem_i)
            pltpu.async_copy(src_hbm.at[pl.ds(_E2 + base, _K2)],
                             idx_b.at[pl.ds(_K2, _K2)], sem_i)
            pltpu.async_copy(dst_hbm.at[pl.ds(base, _K2)],
                             dst_b.at[pl.ds(0, _K2)], sem_i)
            pltpu.async_copy(dst_hbm.at[pl.ds(_E2 + base, _K2)],
                             dst_b.at[pl.ds(_K2, _K2)], sem_i)
            pltpu.async_copy(e_hbm.at[pl.ds(base, _K2)], e_b, sem_i)

    def _wait_prefetch(slot, guard):
        idx_b, dst_b, _, e_b, _, sem_i, _, _ = slot

        @pl.when(guard)
        def _():
            for half in range(2):
                pltpu.make_async_copy(src_hbm.at[pl.ds(0, _K2)],
                                      idx_b.at[pl.ds(half * _K2, _K2)],
                                      sem_i).wait()
                pltpu.make_async_copy(dst_hbm.at[pl.ds(0, _K2)],
                                      dst_b.at[pl.ds(half * _K2, _K2)],
                                      sem_i).wait()
            pltpu.make_async_copy(e_hbm.at[pl.ds(0, _K2)], e_b, sem_i).wait()

    def _wait_scatter(slot, guard):
        _, _, dst2_b, _, xe_b, _, _, sem_s = slot

        @pl.when(guard)
        def _():
            pltpu.make_async_copy(xe_b, acc_sh.at[dst2_b], sem_s).wait()

    def _start_gather(slot, guard):
        idx_b, _, _, _, xe_b, _, sem_g, _ = slot

        @pl.when(guard)
        def _():
            pltpu.async_copy(x_hbm.at[idx_b], xe_b, sem_g)

    def _wait_gather(slot, guard):
        idx_b, _, _, _, xe_b, _, sem_g, _ = slot

        @pl.when(guard)
        def _():
            pltpu.make_async_copy(x_hbm.at[idx_b], xe_b, sem_g).wait()

    def _compute_scatter(slot, guard):
        _, dst_b, dst2_b, e_b, xe_b, _, _, sem_s = slot

        @pl.when(guard)
        def _():
            mask16 = jnp.full((16,), -65536, jnp.int32)
            sh16 = jnp.full((16,), 16, jnp.int32)

            def row(r, rcarry):
                for cc in range(_D // 16):
                    sl = pl.ds(cc * 16, 16)
                    v = e_b[r, sl]
                    ea = lax.bitcast_convert_type(v & mask16, jnp.float32)
                    eb = lax.bitcast_convert_type(lax.shift_left(v, sh16),
                                                  jnp.float32)
                    xe_b[r, sl] = jnp.maximum(xe_b[r, sl] + ea, 0.0)
                    xe_b[r + _K2, sl] = jnp.maximum(xe_b[r + _K2, sl] + eb, 0.0)
                return rcarry
            lax.fori_loop(0, _K2, row, 0)
            for k in range(_K // 16):
                sl = pl.ds(k * 16, 16)
                dst2_b[sl] = dst_b[sl]
            pltpu.async_copy(xe_b, acc_sh.at[dst2_b], sem_s, add=True)

    _prefetch(0, slots[0], 0 < nj)
    _prefetch(1, slots[1], 1 < nj)

    def zrow(r, carry):
        for cc in range(_D // 16):
            xe_v[r, pl.ds(cc * 16, 16)] = jnp.zeros((16,), jnp.float32)
        return carry
    lax.fori_loop(0, _EXPORT_ROWS, zrow, 0)
    for p in range(_PIECES_PER_TILE):
        piece = s + _NS * p

        @pl.when(piece < _NPIECES)
        def _():
            r0 = piece * _EXPORT_ROWS
            pltpu.sync_copy(xe_v.at[pl.ds(0, _EXPORT_ROWS)],
                            acc_sh.at[pl.ds(r0, _EXPORT_ROWS)])
    plsc.subcore_barrier()

    _wait_prefetch(slots[0], 0 < nj)
    _start_gather(slots[0], 0 < nj)

    def pair_body(p, carry):
        for b in range(2):
            j = 2 * p + b
            o = 1 - b
            _wait_prefetch(slots[o], j + 1 < nj)
            _wait_scatter(slots[o], (j >= 1) & (j + 1 < nj))
            _start_gather(slots[o], j + 1 < nj)
            _wait_gather(slots[b], j < nj)
            _compute_scatter(slots[b], j < nj)
            _prefetch(j + 2, slots[b], j + 2 < nj)
        return carry
    lax.fori_loop(0, (_CHUNKS_PER_W + 1) // 2, pair_body, 0)
    _wait_scatter(slots[0], nj >= 1)
    _wait_scatter(slots[1], nj >= 2)
    plsc.subcore_barrier()

    for p in range(_PIECES_PER_TILE):
        piece = s + _NS * p

        @pl.when(piece < _NPIECES)
        def _():
            r0 = piece * _EXPORT_ROWS
            pltpu.sync_copy(acc_sh.at[pl.ds(r0, _EXPORT_ROWS)],
                            xe_v.at[pl.ds(0, _EXPORT_ROWS)])
            pltpu.sync_copy(xe_v.at[pl.ds(0, _EXPORT_ROWS)],
                            out_hbm.at[c, pl.ds(r0, _EXPORT_ROWS)])


def _sc_message(src, dst, e, x):
    mesh = plsc.VectorSubcoreMesh(core_axis_name="c", subcore_axis_name="s")
    f = pl.kernel(
        _sc_message_body,
        out_type=jax.ShapeDtypeStruct((_NC, _N, _D), jnp.float32),
        mesh=mesh,
        scratch_types=(
            [pltpu.VMEM((_K,), jnp.int32)] * 6
            + [pltpu.VMEM((_K, _D), jnp.float32)] * 2
            + [pltpu.VMEM((_K2, _D), jnp.int32)] * 2
            + [pltpu.VMEM_SHARED((_N, _D), jnp.float32)]
            + [pltpu.SemaphoreType.DMA] * 6
        ),
    )
    return f(src, dst, e, x)


def _node_body(inp_ref, agg_ref, w_ref, sc_ref, bi_ref, eps_ref, o_ref):
    z = eps_ref[0, 0] * inp_ref[...] + agg_ref[0] + agg_ref[1]
    h = jnp.dot(z, w_ref[...], preferred_element_type=jnp.float32,
                precision=_HIGH)
    o_ref[...] = _leaky2(h * sc_ref[...] + bi_ref[...])


def _node_update(inp, agg, w, scale, bias, eps1):
    return pl.pallas_call(
        _node_body,
        grid=(_N // _BN,),
        in_specs=[
            pl.BlockSpec((_BN, _D), lambda i: (i, 0)),
            pl.BlockSpec((_NC, _BN, _D), lambda i: (0, i, 0)),
            pl.BlockSpec((_D, _D), lambda i: (0, 0)),
            pl.BlockSpec((1, _D), lambda i: (0, 0)),
            pl.BlockSpec((1, _D), lambda i: (0, 0)),
            pl.BlockSpec(memory_space=pltpu.SMEM),
        ],
        out_specs=pl.BlockSpec((_BN, _D), lambda i: (i, 0)),
        out_shape=jax.ShapeDtypeStruct((_N, _D), jnp.float32),
    )(inp, agg, w, scale, bias, eps1)


def _node_pool_body(inp_ref, agg_ref, w_ref, sc_ref, bi_ref, eps_ref, b_ref,
                    o_ref, po_ref):
    i = pl.program_id(0)
    z = eps_ref[0, 0] * inp_ref[...] + agg_ref[0] + agg_ref[1]
    h = jnp.dot(z, w_ref[...], preferred_element_type=jnp.float32,
                precision=_HIGH)
    h = _leaky2(h * sc_ref[...] + bi_ref[...])
    o_ref[...] = h
    oh_t = (b_ref[0] == lax.broadcasted_iota(jnp.int32, (_G, _BN), 0))
    part = jnp.dot(oh_t.astype(jnp.float32), h,
                   preferred_element_type=jnp.float32, precision=_HIGH)

    @pl.when(i == 0)
    def _():
        po_ref[...] = jnp.zeros_like(po_ref)
    po_ref[...] += part


def _node_pool(inp, agg, w, scale, bias, eps1, batch_row3):
    return pl.pallas_call(
        _node_pool_body,
        grid=(_N // _BN,),
        in_specs=[
            pl.BlockSpec((_BN, _D), lambda i: (i, 0)),
            pl.BlockSpec((_NC, _BN, _D), lambda i: (0, i, 0)),
            pl.BlockSpec((_D, _D), lambda i: (0, 0)),
            pl.BlockSpec((1, _D), lambda i: (0, 0)),
            pl.BlockSpec((1, _D), lambda i: (0, 0)),
            pl.BlockSpec(memory_space=pltpu.SMEM),
            pl.BlockSpec((1, 1, _BN), lambda i: (i, 0, 0)),
        ],
        out_specs=[
            pl.BlockSpec((_BN, _D), lambda i: (i, 0)),
            pl.BlockSpec((_G, _D), lambda i: (0, 0)),
        ],
        out_shape=[
            jax.ShapeDtypeStruct((_N, _D), jnp.float32),
            jax.ShapeDtypeStruct((_G, _D), jnp.float32),
        ],
    )(inp, agg, w, scale, bias, eps1, batch_row3)


def _head_body(h0_ref, h1_ref, b_ref, hp_ref, a0_ref, a1_ref, a2_ref,
               cb0_ref, cw1_ref, cb1_ref, fw_ref, fb_ref, o_ref):
    oh_t = (b_ref[0] == lax.broadcasted_iota(jnp.int32, (_G, _BN), 0))
    hp = lax.dot_general(oh_t.astype(jnp.float32), hp_ref[...],
                         (((0,), (0,)), ((), ())),
                         preferred_element_type=jnp.float32)
    z = (jnp.dot(h0_ref[...], a0_ref[...], preferred_element_type=jnp.float32)
         + jnp.dot(h1_ref[...], a1_ref[...], preferred_element_type=jnp.float32)
         + jnp.dot(hp, a2_ref[...], preferred_element_type=jnp.float32)
         + cb0_ref[...])
    z = jnp.where(z > 0, z, 0.01 * z)
    z = jnp.dot(z, cw1_ref[...], preferred_element_type=jnp.float32) + cb1_ref[...]
    z = jnp.where(z > 0, z, 0.01 * z)
    y = jnp.dot(z, fw_ref[...], preferred_element_type=jnp.float32) + fb_ref[...]
    o_ref[...] = jax.nn.sigmoid(y)


def _head(h0, h1, batch_col3, hp, a0, a1, a2, cb0, cw1, cb1, fw_pad, fb_pad):
    c1 = cw1.shape[0]
    c2 = cw1.shape[1]
    return pl.pallas_call(
        _head_body,
        grid=(_N // _BN,),
        in_specs=[
            pl.BlockSpec((_BN, _D), lambda i: (i, 0)),
            pl.BlockSpec((_BN, _D), lambda i: (i, 0)),
            pl.BlockSpec((1, 1, _BN), lambda i: (i, 0, 0)),
            pl.BlockSpec((_G, _D), lambda i: (0, 0)),
            pl.BlockSpec((_D, c1), lambda i: (0, 0)),
            pl.BlockSpec((_D, c1), lambda i: (0, 0)),
            pl.BlockSpec((_D, c1), lambda i: (0, 0)),
            pl.BlockSpec((1, c1), lambda i: (0, 0)),
            pl.BlockSpec((c1, c2), lambda i: (0, 0)),
            pl.BlockSpec((1, c2), lambda i: (0, 0)),
            pl.BlockSpec((c2, _D), lambda i: (0, 0)),
            pl.BlockSpec((1, _D), lambda i: (0, 0)),
        ],
        out_specs=pl.BlockSpec((_BN, _D), lambda i: (i, 0)),
        out_shape=jax.ShapeDtypeStruct((_N, _D), jnp.float32),
    )(h0, h1, batch_col3, hp, a0, a1, a2, cb0, cw1, cb1, fw_pad, fb_pad)


def kernel(x, edge_index, edge_attr, batch, params):
    src = edge_index[0]
    dst = edge_index[1]

    ea_t = edge_attr.T
    e0 = _edge_transform(ea_t, params['eW0'], params['eb0'][None, :])
    e1 = _edge_transform(ea_t, params['eW1'], params['eb1'][None, :])

    bn = 1.0 / jnp.sqrt(1.0 + 1e-5)
    s0 = (params['g0'] * bn)[None, :]
    t0 = (params['b0'] * params['g0'] * bn + params['be0'])[None, :]
    s1 = (params['g1'] * bn)[None, :]
    t1 = (params['b1'] * params['g1'] * bn + params['be1'])[None, :]
    eps1_0 = (1.0 + params['eps0']).reshape(1, 1)
    eps1_1 = (1.0 + params['eps1']).reshape(1, 1)

    agg0 = _sc_message(src, dst, e0, x)
    h0 = _node_update(x, agg0, params['W0'], s0, t0, eps1_0)
    agg1 = _sc_message(src, dst, e1, h0)
    batch_row3 = batch.reshape(_N // _BN, 1, _BN)
    h1, hp = _node_pool(h0, agg1, params['W1'], s1, t1, eps1_1, batch_row3)

    a0 = params['cW0'][:_D]
    a1 = params['cW0'][_D:2 * _D]
    a2 = params['cW0'][2 * _D:]
    cb0 = params['cb0'][None, :]
    cb1 = params['cb1'][None, :]
    fw_pad = jnp.concatenate(
        [params['fW'], jnp.zeros((params['fW'].shape[0], _D - 1), jnp.float32)],
        axis=1)
    fb_pad = jnp.concatenate(
        [params['fb'], jnp.zeros((_D - 1,), jnp.float32)])[None, :]

    out_pad = _head(h0, h1, batch_row3, hp, a0, a1, a2, cb0,
                    params['cW1'], cb1, fw_pad, fb_pad)
    return out_pad[:, :1]
